# Initial kernel scaffold; baseline (speedup 1.0000x reference)
import jax
import jax.numpy as jnp
from jax.experimental import pallas as pl


def kernel(x, edge_index, edge_attr, batch, conv1, conv2, conv3, lin):
    raise NotImplementedError("write your pallas kernel here")


# speedup vs baseline: 3.5820x; 3.5820x over previous
import functools
import math

import jax
import jax.numpy as jnp
from jax import lax
from jax.experimental import pallas as pl
from jax.experimental.pallas import tpu as pltpu
from jax.experimental.pallas import tpu_sc as plsc

N = 50000
E = 800000
NG = 128
NCLS = 5
HO = 64

NC = 2
NS = 16
NW = NC * NS
RANGE = 1664
NP = NW * RANGE
CH = 128
NCHUNK = RANGE // CH
CAP = 28672
BA = 6400
BG = 512

AVG_DEG_LOG = math.log(17.0)
FMAX = 3.0e38

_SC_PARAMS = pltpu.CompilerParams(needs_layout_passes=False,
                                  use_tc_tiling_on_sc=False)


@functools.cache
def _mesh():
    return plsc.VectorSubcoreMesh(core_axis_name="c", subcore_axis_name="s")


def _wid():
    return lax.axis_index("s") * NC + lax.axis_index("c")


def _splat(v):
    return jnp.full((16,), v, jnp.int32)


def _sget(ref, i):
    return plsc.load_gather(ref, indices=[_splat(i)])[0]


def _sset(ref, i, v):
    plsc.store_scatter(ref, [_splat(i)], jnp.full((16,), v))


def _bucket_body(dst_h, src_h, ea_h, pidx_h, eas_h, choff_h, deg_h,
                 dbuf, sbuf, ebuf, stg_p, stg_e, hist, offs, pos2, degf,
                 chof, sem):
    wid = _wid()
    lo = wid * RANGE

    def z16(v, _):
        stg_p[pl.ds(v * 16, 16)] = jnp.zeros((16,), jnp.int32)
        return 0
    lax.fori_loop(0, CAP // 16, z16, 0)

    def zh(v, _):
        hist[pl.ds(v * 16, 16)] = jnp.zeros((16,), jnp.int32)
        return 0
    lax.fori_loop(0, RANGE // 16, zh, 0)

    def zp(v, _):
        pos2[v >> 3, pl.ds((v & 7) * 16, 16)] = (
            lax.iota(jnp.int32, 16) + v * 16 + 128)
        return 0
    lax.fori_loop(0, CAP // 16, zp, 0)

    def blk(b, cur):
        base = b * BA
        pltpu.sync_copy(dst_h.at[pl.ds(base, BA)], dbuf)
        pltpu.sync_copy(src_h.at[pl.ds(base, BA)], sbuf)
        pltpu.sync_copy(ea_h.at[pl.ds(base, BA)], ebuf)

        def vec(i, cur):
            d = dbuf[pl.ds(i * 16, 16)]
            s = sbuf[pl.ds(i * 16, 16)]
            a = ebuf[pl.ds(i * 16, 16)]
            msk = (d >= lo) & (d < lo + RANGE)
            pk = ((d - lo) << 16) | s
            plsc.store_compressed(stg_p.at[pl.ds(cur, 16)], pk, mask=msk)
            plsc.store_compressed(stg_e.at[pl.ds(cur, 16)], a, mask=msk)
            return cur + jnp.sum(msk.astype(jnp.int32))
        return lax.fori_loop(0, BA // 16, vec, cur)
    cnt = lax.fori_loop(0, E // BA, blk, jnp.int32(0))

    def h1(e, _):
        ld = stg_p[pl.ds(e, 16)][0] >> 16
        _sset(hist, ld, _sget(hist, ld) + 1)
        return 0
    lax.fori_loop(0, cnt, h1, 0)

    def pre(cc, run):
        start = (run + 7) & ~7
        _sset(chof, cc, start)

        def v8(v, run):
            sl = pl.ds(cc * CH + v * 16, 16)
            hv = hist[sl]
            offs[sl] = plsc.cumsum(hv) - hv + run
            return run + jnp.sum(hv)
        run = lax.fori_loop(0, CH // 16, v8, start)
        _sset(chof, 16 + cc, run - start)
        return run
    lax.fori_loop(0, NCHUNK, pre, jnp.int32(0))
    _sset(chof, 15, cnt)

    def dcv(v, _):
        degf[pl.ds(v * 16, 16)] = hist[pl.ds(v * 16, 16)].astype(jnp.float32)
        return 0
    lax.fori_loop(0, RANGE // 16, dcv, 0)
    pltpu.sync_copy(degf, deg_h.at[pl.ds(lo, RANGE)])
    pltpu.sync_copy(chof, choff_h.at[wid])

    def p3(e, _):
        ld = stg_p[pl.ds(e, 16)][0] >> 16
        p = _sget(offs, ld)
        _sset(offs, ld, p + 1)
        plsc.store_scatter(pos2, [_splat(e >> 7), _splat(e & 127)],
                           jnp.full((16,), p))
        return 0
    lax.fori_loop(0, cnt, p3, 0)

    nrows = (cnt + 127) >> 7

    def srow(j, _):
        c1 = pltpu.async_copy(
            stg_p.at[pl.ds(j * 128, 128)],
            pidx_h.at[wid].at[pos2.at[j]], sem)
        c2 = pltpu.async_copy(
            stg_e.at[pl.ds(j * 128, 128)],
            eas_h.at[wid].at[pos2.at[j]], sem)
        c1.wait()
        c2.wait()
        return 0
    lax.fori_loop(0, nrows, srow, 0)


def _bucket(dst, src, ea):
    f = pl.kernel(
        _bucket_body,
        out_type=(
            jax.ShapeDtypeStruct((NW, CAP), jnp.int32),
            jax.ShapeDtypeStruct((NW, CAP), jnp.float32),
            jax.ShapeDtypeStruct((NW, 32), jnp.int32),
            jax.ShapeDtypeStruct((NP,), jnp.float32),
        ),
        mesh=_mesh(),
        compiler_params=_SC_PARAMS,
        scratch_types=[
            pltpu.VMEM((BA,), jnp.int32),
            pltpu.VMEM((BA,), jnp.int32),
            pltpu.VMEM((BA,), jnp.float32),
            pltpu.VMEM((CAP + 16,), jnp.int32),
            pltpu.VMEM((CAP + 16,), jnp.float32),
            pltpu.VMEM((RANGE,), jnp.int32),
            pltpu.VMEM((RANGE,), jnp.int32),
            pltpu.VMEM((CAP // 128, 128), jnp.int32),
            pltpu.VMEM((RANGE,), jnp.float32),
            pltpu.VMEM((32,), jnp.int32),
            pltpu.SemaphoreType.DMA,
        ],
    )
    return f(dst, src, ea)


def _edge_body(wf, bp_h, ap_h, pidx_h, eas_h, choff_h, w_h,
               ssum_h, ssq_h, smn_h, smx_h,
               pbuf, ebuf, idx2, rows, au, asum, asq, amn, amx, wv, chof,
               sem):
    wid = _wid()
    nb = wid * RANGE
    pltpu.sync_copy(choff_h.at[wid], chof)
    pltpu.sync_copy(w_h, wv)
    nj = wf // 16

    def chunk(c, _):
        e0 = _sget(chof, c)
        ec = _sget(chof, 16 + c)
        nbase = nb + c * CH

        def init_row(r, _):
            for j in range(nj):
                sl = pl.ds(j * 16, 16)
                asum[r, sl] = jnp.zeros((16,), jnp.float32)
                asq[r, sl] = jnp.zeros((16,), jnp.float32)
                amn[r, sl] = jnp.full((16,), FMAX, jnp.float32)
                amx[r, sl] = jnp.full((16,), -FMAX, jnp.float32)
            return 0
        lax.fori_loop(0, CH, init_row, 0)

        pltpu.sync_copy(ap_h.at[pl.ds(nbase, CH)], au)

        nblk = (ec + BG - 1) >> 9

        def blk(b, _):
            base = pl.multiple_of(e0 + b * BG, 8)
            pltpu.sync_copy(pidx_h.at[wid].at[pl.ds(base, BG)],
                            pbuf.at[pl.ds(0, BG)])
            pltpu.sync_copy(eas_h.at[wid].at[pl.ds(base, BG)],
                            ebuf.at[pl.ds(0, BG)])

            def vi(v, _):
                s = pbuf[pl.ds(v * 16, 16)] & 0xFFFF
                idx2[v >> 3, pl.ds((v & 7) * 16, 16)] = jnp.minimum(
                    s, jnp.int32(N - 1))
                return 0
            lax.fori_loop(0, BG // 16, vi, 0)

            cps = [pltpu.async_copy(bp_h.at[idx2.at[j]],
                                    rows.at[pl.ds(j * 128, 128)], sem)
                   for j in range(BG // 128)]
            for cp in cps:
                cp.wait()

            nrem = jnp.minimum(jnp.int32(BG), ec - b * BG)

            def edge(e, _):
                pk = pbuf[pl.ds(e, 16)][0]
                lc = (pk >> 16) - c * CH
                a = ebuf[pl.ds(e, 16)][0]
                for j in range(nj):
                    sl = pl.ds(j * 16, 16)
                    t = rows[e, sl] + a * wv[sl] + au[lc, sl]
                    plsc.addupdate(asum.at[lc, sl], t)
                    plsc.addupdate(asq.at[lc, sl], t * t)
                    amn[lc, sl] = jnp.minimum(amn[lc, sl], t)
                    amx[lc, sl] = jnp.maximum(amx[lc, sl], t)
                return 0
            lax.fori_loop(0, nrem, edge, 0)
            return 0
        lax.fori_loop(0, nblk, blk, 0)

        pltpu.sync_copy(asum, ssum_h.at[pl.ds(nbase, CH)])
        pltpu.sync_copy(asq, ssq_h.at[pl.ds(nbase, CH)])
        pltpu.sync_copy(amn, smn_h.at[pl.ds(nbase, CH)])
        pltpu.sync_copy(amx, smx_h.at[pl.ds(nbase, CH)])
        return 0
    lax.fori_loop(0, NCHUNK, chunk, 0)


def _edge_stats(bp, ap, pidx, eas, choff, w, wf):
    f = pl.kernel(
        functools.partial(_edge_body, wf),
        out_type=(
            jax.ShapeDtypeStruct((NP, wf), jnp.float32),
            jax.ShapeDtypeStruct((NP, wf), jnp.float32),
            jax.ShapeDtypeStruct((NP, wf), jnp.float32),
            jax.ShapeDtypeStruct((NP, wf), jnp.float32),
        ),
        mesh=_mesh(),
        compiler_params=_SC_PARAMS,
        scratch_types=[
            pltpu.VMEM((BG + 16,), jnp.int32),
            pltpu.VMEM((BG + 16,), jnp.float32),
            pltpu.VMEM((BG // 128, 128), jnp.int32),
            pltpu.VMEM((BG, wf), jnp.float32),
            pltpu.VMEM((CH, wf), jnp.float32),
            pltpu.VMEM((CH, wf), jnp.float32),
            pltpu.VMEM((CH, wf), jnp.float32),
            pltpu.VMEM((CH, wf), jnp.float32),
            pltpu.VMEM((CH, wf), jnp.float32),
            pltpu.VMEM((wf,), jnp.float32),
            pltpu.VMEM((32,), jnp.int32),
            pltpu.SemaphoreType.DMA,
        ],
    )
    return f(bp, ap, pidx, eas, choff, w)


NB = 256
NGRID = NP // NB


def _pre1_body(x_ref, wi_ref, wj_ref, c_ref, a_ref, b_ref):
    x = x_ref[...]
    a_ref[...] = jnp.dot(x, wi_ref[...],
                         preferred_element_type=jnp.float32) + c_ref[...]
    b_ref[...] = jnp.dot(x, wj_ref[...], preferred_element_type=jnp.float32)


def _pre1(xp, wi, wj, c):
    wf = wi.shape[1]
    return pl.pallas_call(
        _pre1_body,
        grid=(NGRID,),
        in_specs=[
            pl.BlockSpec((NB, xp.shape[1]), lambda i: (i, 0)),
            pl.BlockSpec((wi.shape[0], wf), lambda i: (0, 0)),
            pl.BlockSpec((wj.shape[0], wf), lambda i: (0, 0)),
            pl.BlockSpec((1, wf), lambda i: (0, 0)),
        ],
        out_specs=[
            pl.BlockSpec((NB, wf), lambda i: (i, 0)),
            pl.BlockSpec((NB, wf), lambda i: (i, 0)),
        ],
        out_shape=[
            jax.ShapeDtypeStruct((NP, wf), jnp.float32),
            jax.ShapeDtypeStruct((NP, wf), jnp.float32),
        ],
    )(xp, wi, wj, c)


def _post_math(h, ssum, ssq, smn, smx, d, wpp, bpost, wlin, blin, wf):
    degc = jnp.maximum(d, 1.0)
    mean = ssum / degc
    var = ssq / degc - mean * mean
    std = jnp.sqrt(jnp.maximum(var, 0.0) + 1e-5)
    nz = (d > 0.0).astype(jnp.float32)
    mn = smn * nz
    mx = smx * nz
    logd = jnp.log(degc + 1.0)
    s2 = logd * (1.0 / AVG_DEG_LOG)
    s3 = AVG_DEG_LOG / logd
    o = jnp.dot(h, wpp[0:wf], preferred_element_type=jnp.float32)
    for k, p in enumerate((mean, mn, mx, std)):
        o += jnp.dot(p, wpp[(1 + k) * wf:(2 + k) * wf],
                     preferred_element_type=jnp.float32)
        o += jnp.dot(p * s2, wpp[(5 + k) * wf:(6 + k) * wf],
                     preferred_element_type=jnp.float32)
        o += jnp.dot(p * s3, wpp[(9 + k) * wf:(10 + k) * wf],
                     preferred_element_type=jnp.float32)
    o = o + bpost
    return jnp.dot(o, wlin, preferred_element_type=jnp.float32) + blin


def _post_body(wf, h_ref, ssum_ref, ssq_ref, smn_ref, smx_ref, d_ref,
               wpp_ref, bpost_ref, wlin_ref, blin_ref,
               win_ref, cn_ref, wjn_ref,
               h2_ref, an_ref, bn_ref):
    o = _post_math(h_ref[...], ssum_ref[...], ssq_ref[...], smn_ref[...],
                   smx_ref[...], d_ref[...], wpp_ref[...], bpost_ref[...],
                   wlin_ref[...], blin_ref[...], wf)
    o = jnp.where(o > 0, o, 0.01 * o)
    h2_ref[...] = o
    an_ref[...] = jnp.dot(o, win_ref[...],
                          preferred_element_type=jnp.float32) + cn_ref[...]
    bn_ref[...] = jnp.dot(o, wjn_ref[...],
                          preferred_element_type=jnp.float32)


def _post_pre(h, ssum, ssq, smn, smx, deg2, wpp, bpost, wlin, blin,
              win, cn, wjn, wf):
    wfn = win.shape[1]
    return pl.pallas_call(
        functools.partial(_post_body, wf),
        grid=(NGRID,),
        in_specs=[
            pl.BlockSpec((NB, h.shape[1]), lambda i: (i, 0)),
            pl.BlockSpec((NB, wf), lambda i: (i, 0)),
            pl.BlockSpec((NB, wf), lambda i: (i, 0)),
            pl.BlockSpec((NB, wf), lambda i: (i, 0)),
            pl.BlockSpec((NB, wf), lambda i: (i, 0)),
            pl.BlockSpec((NB, 1), lambda i: (i, 0)),
            pl.BlockSpec(wpp.shape, lambda i: (0, 0)),
            pl.BlockSpec((1, HO), lambda i: (0, 0)),
            pl.BlockSpec((HO, HO), lambda i: (0, 0)),
            pl.BlockSpec((1, HO), lambda i: (0, 0)),
            pl.BlockSpec((HO, wfn), lambda i: (0, 0)),
            pl.BlockSpec((1, wfn), lambda i: (0, 0)),
            pl.BlockSpec((HO, wfn), lambda i: (0, 0)),
        ],
        out_specs=[
            pl.BlockSpec((NB, HO), lambda i: (i, 0)),
            pl.BlockSpec((NB, wfn), lambda i: (i, 0)),
            pl.BlockSpec((NB, wfn), lambda i: (i, 0)),
        ],
        out_shape=[
            jax.ShapeDtypeStruct((NP, HO), jnp.float32),
            jax.ShapeDtypeStruct((NP, wfn), jnp.float32),
            jax.ShapeDtypeStruct((NP, wfn), jnp.float32),
        ],
    )(h, ssum, ssq, smn, smx, deg2, wpp, bpost, wlin, blin, win, cn, wjn)


def _final_body(wf, h_ref, ssum_ref, ssq_ref, smn_ref, smx_ref, d_ref,
                wpp_ref, bpost_ref, wlin_ref, blin_ref, bt_ref,
                w1_ref, b1_ref, w2_ref, b2_ref,
                out_ref, pacc, cacc):
    i = pl.program_id(0)

    @pl.when(i == 0)
    def _():
        pacc[...] = jnp.zeros_like(pacc)
        cacc[...] = jnp.zeros_like(cacc)

    o = _post_math(h_ref[...], ssum_ref[...], ssq_ref[...], smn_ref[...],
                   smx_ref[...], d_ref[...], wpp_ref[...], bpost_ref[...],
                   wlin_ref[...], blin_ref[...], wf)
    bt = bt_ref[...]
    oh = (bt == lax.broadcasted_iota(jnp.int32, (NB, NG), 1)).astype(
        jnp.float32)
    pacc[...] += lax.dot_general(oh, o, (((0,), (0,)), ((), ())),
                                 preferred_element_type=jnp.float32)
    cacc[...] += lax.dot_general(
        oh, jnp.ones((NB, 8), jnp.float32), (((0,), (0,)), ((), ())),
        preferred_element_type=jnp.float32)

    @pl.when(i == NGRID - 1)
    def _():
        cnt = jnp.maximum(cacc[...][:, 0:1], 1.0)
        pooled = pacc[...] / cnt
        z = jnp.dot(pooled, w1_ref[...],
                    preferred_element_type=jnp.float32) + b1_ref[...]
        z = jnp.maximum(z, 0.0)
        out_ref[...] = jnp.dot(z, w2_ref[...],
                               preferred_element_type=jnp.float32) + b2_ref[...]


def _final(h, ssum, ssq, smn, smx, deg2, wpp, bpost, wlin, blin, batch2,
           w1, b1, w2p, b2p, wf):
    return pl.pallas_call(
        functools.partial(_final_body, wf),
        grid=(NGRID,),
        in_specs=[
            pl.BlockSpec((NB, h.shape[1]), lambda i: (i, 0)),
            pl.BlockSpec((NB, wf), lambda i: (i, 0)),
            pl.BlockSpec((NB, wf), lambda i: (i, 0)),
            pl.BlockSpec((NB, wf), lambda i: (i, 0)),
            pl.BlockSpec((NB, wf), lambda i: (i, 0)),
            pl.BlockSpec((NB, 1), lambda i: (i, 0)),
            pl.BlockSpec(wpp.shape, lambda i: (0, 0)),
            pl.BlockSpec((1, HO), lambda i: (0, 0)),
            pl.BlockSpec((HO, HO), lambda i: (0, 0)),
            pl.BlockSpec((1, HO), lambda i: (0, 0)),
            pl.BlockSpec((NB, 1), lambda i: (i, 0)),
            pl.BlockSpec((HO, 32), lambda i: (0, 0)),
            pl.BlockSpec((1, 32), lambda i: (0, 0)),
            pl.BlockSpec((32, 128), lambda i: (0, 0)),
            pl.BlockSpec((1, 128), lambda i: (0, 0)),
        ],
        out_specs=[pl.BlockSpec((NG, 128), lambda i: (0, 0))],
        out_shape=[jax.ShapeDtypeStruct((NG, 128), jnp.float32)],
        scratch_shapes=[
            pltpu.VMEM((NG, HO), jnp.float32),
            pltpu.VMEM((NG, 8), jnp.float32),
        ],
    )(h, ssum, ssq, smn, smx, deg2, wpp, bpost, wlin, blin, batch2,
      w1, b1, w2p, b2p)[0]


def _prep_conv(p, f_real, wf):
    wpre = p["Wpre"]
    wi = wpre[:f_real]
    wj = wpre[f_real:2 * f_real]
    we2 = wpre[2 * f_real:3 * f_real]
    w = p["We"][0] @ we2
    c = p["be"] @ we2 + p["bpre"]
    pad = wf - f_real
    wi = jnp.pad(wi, ((0, pad), (0, pad)))
    wj = jnp.pad(wj, ((0, pad), (0, pad)))
    w = jnp.pad(w, (0, pad))
    c = jnp.pad(c, (0, pad))
    wpost = p["Wpost"]
    blocks = [jnp.pad(wpost[k * f_real:(k + 1) * f_real], ((0, pad), (0, 0)))
              for k in range(13)]
    wpp = jnp.concatenate(blocks, axis=0)
    return (wi, wj, c[None, :], w, wpp, p["bpost"][None, :],
            p["Wlin"], p["blin"][None, :])


def kernel(x, edge_index, edge_attr, batch, conv1, conv2, conv3, lin):
    src = edge_index[0].astype(jnp.int32)
    dst = edge_index[1].astype(jnp.int32)
    ea = edge_attr[:, 0]

    wi1, wj1, c1, w1v, wpp1, bp1, wl1, bl1 = _prep_conv(conv1, 7, 16)
    wi2, wj2, c2, w2v, wpp2, bp2, wl2, bl2 = _prep_conv(conv2, 64, 64)
    wi3, wj3, c3, w3v, wpp3, bp3, wl3, bl3 = _prep_conv(conv3, 64, 64)

    xp = jnp.pad(x, ((0, NP - N), (0, 16 - 7)))
    batch2 = jnp.pad(batch.astype(jnp.int32), (0, NP - N),
                     constant_values=NG)[:, None]
    w2p = jnp.pad(lin["W2"], ((0, 0), (0, 128 - NCLS)))
    b2p = jnp.pad(lin["b2"], (0, 128 - NCLS))[None, :]

    pidx, eas, choff, deg = _bucket(dst, src, ea)
    deg2 = deg[:, None]

    a1, b1 = _pre1(xp, wi1, wj1, c1)
    s1, q1, mn1, mx1 = _edge_stats(b1, a1, pidx, eas, choff, w1v, 16)
    h2, a2, b2 = _post_pre(xp, s1, q1, mn1, mx1, deg2, wpp1, bp1, wl1, bl1,
                           wi2, c2, wj2, 16)
    s2, q2, mn2, mx2 = _edge_stats(b2, a2, pidx, eas, choff, w2v, 64)
    h3, a3, b3 = _post_pre(h2, s2, q2, mn2, mx2, deg2, wpp2, bp2, wl2, bl2,
                           wi3, c3, wj3, 64)
    s3, q3, mn3, mx3 = _edge_stats(b3, a3, pidx, eas, choff, w3v, 64)
    out = _final(h3, s3, q3, mn3, mx3, deg2, wpp3, bp3, wl3, bl3, batch2,
                 lin["W1"], lin["b1"][None, :], w2p, b2p, 64)
    return out[:, :NCLS]


# speedup vs baseline: 4.6366x; 1.2944x over previous
import functools
import math

import jax
import jax.numpy as jnp
from jax import lax
from jax.experimental import pallas as pl
from jax.experimental.pallas import tpu as pltpu
from jax.experimental.pallas import tpu_sc as plsc

N = 50000
E = 800000
NG = 128
NCLS = 5
HO = 64

NC = 2
NS = 16
NW = NC * NS
RANGE = 1664
NP = NW * RANGE
CH = 128
NCHUNK = RANGE // CH
CAP = 28672
BA = 6400
BG = 512

AVG_DEG_LOG = math.log(17.0)
FMAX = 3.0e38

_SC_PARAMS = pltpu.CompilerParams(needs_layout_passes=False,
                                  use_tc_tiling_on_sc=False)


@functools.cache
def _mesh():
    return plsc.VectorSubcoreMesh(core_axis_name="c", subcore_axis_name="s")


def _wid():
    return lax.axis_index("s") * NC + lax.axis_index("c")


def _splat(v):
    return jnp.full((16,), v, jnp.int32)


def _sget(ref, i):
    return plsc.load_gather(ref, indices=[_splat(i)])[0]


def _sset(ref, i, v):
    plsc.store_scatter(ref, [_splat(i)], jnp.full((16,), v))


def _bucket_body(dst_h, src_h, ea_h, pidx_h, eas_h, choff_h, deg_h,
                 dbuf, sbuf, ebuf, stg_p, stg_e, hist16, off16, deg16, pos2,
                 degf, chof, sem):
    wid = _wid()
    lo = wid * RANGE
    lanes = lax.iota(jnp.int32, 16)
    ones16 = jnp.ones((16,), jnp.int32)

    def z16(v, _):
        stg_p[pl.ds(v * 16, 16)] = jnp.zeros((16,), jnp.int32)
        return 0
    lax.fori_loop(0, CAP // 16, z16, 0)

    def zh(v, _):
        hist16[pl.ds(v * 16, 16)] = jnp.zeros((16,), jnp.int32)
        return 0
    lax.fori_loop(0, 224 // 16, zh, 0)

    def zp(v, _):
        pos2[v >> 3, pl.ds((v & 7) * 16, 16)] = (
            lax.iota(jnp.int32, 16) + v * 16 + 128)
        return 0
    lax.fori_loop(0, CAP // 16, zp, 0)

    def blk(b, cur):
        base = b * BA
        pltpu.sync_copy(dst_h.at[pl.ds(base, BA)], dbuf)
        pltpu.sync_copy(src_h.at[pl.ds(base, BA)], sbuf)
        pltpu.sync_copy(ea_h.at[pl.ds(base, BA)], ebuf)

        def vec(i, cur):
            d = dbuf[pl.ds(i * 16, 16)]
            s = sbuf[pl.ds(i * 16, 16)]
            a = ebuf[pl.ds(i * 16, 16)]
            msk = (d >= lo) & (d < lo + RANGE)
            pk = ((d - lo) << 16) | s
            plsc.store_compressed(stg_p.at[pl.ds(cur, 16)], pk, mask=msk)
            plsc.store_compressed(stg_e.at[pl.ds(cur, 16)], a, mask=msk)
            return cur + jnp.sum(msk.astype(jnp.int32))
        return lax.fori_loop(0, BA // 16, vec, cur)
    cnt = lax.fori_loop(0, E // BA, blk, jnp.int32(0))

    nfull = cnt >> 4
    tail = cnt & 15
    tm = lanes < tail

    def hv(v, _):
        pk = stg_p[pl.ds(v * 16, 16)]
        plsc.addupdate_scatter(hist16, [(pk >> 23) * 16 + lanes], ones16)
        return 0
    lax.fori_loop(0, nfull, hv, 0)
    pkt = stg_p[pl.ds(nfull * 16, 16)]
    plsc.addupdate_scatter(hist16, [(pkt >> 23) * 16 + lanes], ones16,
                           mask=tm)

    def pre(cc, run):
        start = (run + 7) & ~7
        _sset(chof, cc, start)
        sl = pl.ds(cc * 16, 16)
        hv16 = hist16[sl]
        off16[sl] = plsc.cumsum(hv16) - hv16 + start
        tot = jnp.sum(hv16)
        _sset(chof, 16 + cc, tot)
        return start + tot
    lax.fori_loop(0, NCHUNK, pre, jnp.int32(0))
    _sset(chof, 15, cnt)
    pltpu.sync_copy(chof, choff_h.at[wid])

    def qv(v, _):
        pk = stg_p[pl.ds(v * 16, 16)]
        idxv = (pk >> 23) * 16 + lanes
        p = plsc.load_gather(off16, [idxv])
        plsc.store_scatter(off16, [idxv], p + 1)
        pos2[v >> 3, pl.ds((v & 7) * 16, 16)] = p
        return 0
    lax.fori_loop(0, nfull, qv, 0)
    idxt = (pkt >> 23) * 16 + lanes
    pt = plsc.load_gather(off16, [idxt], mask=tm)
    plsc.store_scatter(off16, [idxt], pt + 1, mask=tm)
    tsl = pl.ds((nfull * 16) & 127, 16)
    pos2[nfull >> 3, tsl] = jnp.where(tm, pt, pos2[nfull >> 3, tsl])

    def dchunk(c, _):
        def zd(v, _):
            deg16[pl.ds(v * 16, 16)] = jnp.zeros((16,), jnp.int32)
            return 0
        lax.fori_loop(0, CH, zd, 0)

        def sv(v, _):
            pk = stg_p[pl.ds(v * 16, 16)]
            valid = (v * 16 + lanes) < cnt
            msk = ((pk >> 23) == c) & valid
            idxv = ((pk >> 16) - c * CH) * 16 + lanes
            idxv = jnp.clip(idxv, 0, CH * 16 - 1)
            plsc.addupdate_scatter(deg16, [idxv], ones16, mask=msk)
            return 0
        lax.fori_loop(0, (cnt + 15) >> 4, sv, 0)

        def rd(n, _):
            s = jnp.sum(deg16[pl.ds(n * 16, 16)])
            _sset(degf, c * CH + n, s.astype(jnp.float32))
            return 0
        lax.fori_loop(0, CH, rd, 0)
        return 0
    lax.fori_loop(0, NCHUNK, dchunk, 0)
    pltpu.sync_copy(degf, deg_h.at[pl.ds(lo, RANGE)])

    nrows = (cnt + 127) >> 7

    def srow(j, _):
        c1 = pltpu.async_copy(
            stg_p.at[pl.ds(j * 128, 128)],
            pidx_h.at[wid].at[pos2.at[j]], sem)
        c2 = pltpu.async_copy(
            stg_e.at[pl.ds(j * 128, 128)],
            eas_h.at[wid].at[pos2.at[j]], sem)
        c1.wait()
        c2.wait()
        return 0
    lax.fori_loop(0, nrows, srow, 0)


def _bucket(dst, src, ea):
    f = pl.kernel(
        _bucket_body,
        out_type=(
            jax.ShapeDtypeStruct((NW, CAP), jnp.int32),
            jax.ShapeDtypeStruct((NW, CAP), jnp.float32),
            jax.ShapeDtypeStruct((NW, 32), jnp.int32),
            jax.ShapeDtypeStruct((NP,), jnp.float32),
        ),
        mesh=_mesh(),
        compiler_params=_SC_PARAMS,
        scratch_types=[
            pltpu.VMEM((BA,), jnp.int32),
            pltpu.VMEM((BA,), jnp.int32),
            pltpu.VMEM((BA,), jnp.float32),
            pltpu.VMEM((CAP + 16,), jnp.int32),
            pltpu.VMEM((CAP + 16,), jnp.float32),
            pltpu.VMEM((224,), jnp.int32),
            pltpu.VMEM((224,), jnp.int32),
            pltpu.VMEM((CH * 16,), jnp.int32),
            pltpu.VMEM((CAP // 128, 128), jnp.int32),
            pltpu.VMEM((RANGE,), jnp.float32),
            pltpu.VMEM((32,), jnp.int32),
            pltpu.SemaphoreType.DMA,
        ],
    )
    return f(dst, src, ea)


def _edge_body(wf, bp_h, pidx_h, eas_h, choff_h, w_h,
               ssum_h, ssq_h, smn_h, smx_h,
               pb0, pb1, eb0, eb1, ix0, ix1, rw0, rw1,
               asum, asq, amn, amx, wv, chof,
               smi0, smi1, smg0, smg1):
    wid = _wid()
    nb = wid * RANGE
    pltpu.sync_copy(choff_h.at[wid], chof)
    pltpu.sync_copy(w_h, wv)
    nj = wf // 16
    pbufs = (pb0, pb1)
    ebufs = (eb0, eb1)
    ixs = (ix0, ix1)
    rws = (rw0, rw1)
    smis = (smi0, smi1)
    smgs = (smg0, smg1)

    def chunk(c, _):
        e0 = _sget(chof, c)
        ec = _sget(chof, 16 + c)
        nbase = nb + c * CH

        def init_row(r, _):
            for j in range(nj):
                sl = pl.ds(j * 16, 16)
                asum[r, sl] = jnp.zeros((16,), jnp.float32)
                asq[r, sl] = jnp.zeros((16,), jnp.float32)
                amn[r, sl] = jnp.full((16,), FMAX, jnp.float32)
                amx[r, sl] = jnp.full((16,), -FMAX, jnp.float32)
            return 0
        lax.fori_loop(0, CH, init_row, 0)

        nblk = (ec + BG - 1) >> 9

        def in_descs(b, par):
            base = pl.multiple_of(e0 + b * BG, 8)
            d1 = pltpu.make_async_copy(pidx_h.at[wid].at[pl.ds(base, BG)],
                                       pbufs[par].at[pl.ds(0, BG)], smis[par])
            d2 = pltpu.make_async_copy(eas_h.at[wid].at[pl.ds(base, BG)],
                                       ebufs[par].at[pl.ds(0, BG)], smis[par])
            return d1, d2

        def g_descs(par):
            return [pltpu.make_async_copy(bp_h.at[ixs[par].at[j]],
                                          rws[par].at[pl.ds(j * 128, 128)],
                                          smgs[par])
                    for j in range(BG // 128)]

        def fire_in(b, par):
            d1, d2 = in_descs(b, par)
            d1.start()
            d2.start()

        def wait_in(b, par):
            d1, d2 = in_descs(b, par)
            d1.wait()
            d2.wait()

        def idx_and_gather(par):
            def vi(v, _):
                s = pbufs[par][pl.ds(v * 16, 16)] & 0xFFFF
                ixs[par][v >> 3, pl.ds((v & 7) * 16, 16)] = jnp.minimum(
                    s, jnp.int32(N - 1))
                return 0
            lax.fori_loop(0, BG // 16, vi, 0)
            for d in g_descs(par):
                d.start()

        def edges(b, par):
            nrem = jnp.minimum(jnp.int32(BG), ec - b * BG)
            pbuf, ebuf, rows = pbufs[par], ebufs[par], rws[par]

            def edge(e, _):
                pk = pbuf[pl.ds(e, 16)][0]
                lc = (pk >> 16) - c * CH
                a = ebuf[pl.ds(e, 16)][0]
                for j in range(nj):
                    sl = pl.ds(j * 16, 16)
                    t = rows[e, sl] + a * wv[sl]
                    plsc.addupdate(asum.at[lc, sl], t)
                    plsc.addupdate(asq.at[lc, sl], t * t)
                    amn[lc, sl] = jnp.minimum(amn[lc, sl], t)
                    amx[lc, sl] = jnp.maximum(amx[lc, sl], t)
                return 0
            lax.fori_loop(0, nrem, edge, 0)

        @pl.when(nblk > 0)
        def _():
            fire_in(0, 0)
            wait_in(0, 0)
            idx_and_gather(0)

        @pl.when(nblk > 1)
        def _():
            fire_in(1, 1)

        def pair(q, _):
            for par in (0, 1):
                b = 2 * q + par

                @pl.when(b < nblk)
                def _():
                    @pl.when(b + 1 < nblk)
                    def _():
                        wait_in(b + 1, 1 - par)
                        idx_and_gather(1 - par)
                    for d in g_descs(par):
                        d.wait()
                    edges(b, par)

                    @pl.when(b + 2 < nblk)
                    def _():
                        fire_in(b + 2, par)
            return 0
        lax.fori_loop(0, (nblk + 1) >> 1, pair, 0)

        pltpu.sync_copy(asum, ssum_h.at[pl.ds(nbase, CH)])
        pltpu.sync_copy(asq, ssq_h.at[pl.ds(nbase, CH)])
        pltpu.sync_copy(amn, smn_h.at[pl.ds(nbase, CH)])
        pltpu.sync_copy(amx, smx_h.at[pl.ds(nbase, CH)])
        return 0
    lax.fori_loop(0, NCHUNK, chunk, 0)


def _edge_stats(bp, pidx, eas, choff, w, wf):
    f = pl.kernel(
        functools.partial(_edge_body, wf),
        out_type=(
            jax.ShapeDtypeStruct((NP, wf), jnp.float32),
            jax.ShapeDtypeStruct((NP, wf), jnp.float32),
            jax.ShapeDtypeStruct((NP, wf), jnp.float32),
            jax.ShapeDtypeStruct((NP, wf), jnp.float32),
        ),
        mesh=_mesh(),
        compiler_params=_SC_PARAMS,
        scratch_types=[
            pltpu.VMEM((BG + 16,), jnp.int32),
            pltpu.VMEM((BG + 16,), jnp.int32),
            pltpu.VMEM((BG + 16,), jnp.float32),
            pltpu.VMEM((BG + 16,), jnp.float32),
            pltpu.VMEM((BG // 128, 128), jnp.int32),
            pltpu.VMEM((BG // 128, 128), jnp.int32),
            pltpu.VMEM((BG, wf), jnp.float32),
            pltpu.VMEM((BG, wf), jnp.float32),
            pltpu.VMEM((CH, wf), jnp.float32),
            pltpu.VMEM((CH, wf), jnp.float32),
            pltpu.VMEM((CH, wf), jnp.float32),
            pltpu.VMEM((CH, wf), jnp.float32),
            pltpu.VMEM((wf,), jnp.float32),
            pltpu.VMEM((32,), jnp.int32),
            pltpu.SemaphoreType.DMA,
            pltpu.SemaphoreType.DMA,
            pltpu.SemaphoreType.DMA,
            pltpu.SemaphoreType.DMA,
        ],
    )
    return f(bp, pidx, eas, choff, w)


NB = 256
NGRID = NP // NB


def _pre1_body(x_ref, wi_ref, wj_ref, c_ref, a_ref, b_ref):
    x = x_ref[...]
    a_ref[...] = jnp.dot(x, wi_ref[...],
                         preferred_element_type=jnp.float32) + c_ref[...]
    b_ref[...] = jnp.dot(x, wj_ref[...], preferred_element_type=jnp.float32)


def _pre1(xp, wi, wj, c):
    wf = wi.shape[1]
    return pl.pallas_call(
        _pre1_body,
        grid=(NGRID,),
        in_specs=[
            pl.BlockSpec((NB, xp.shape[1]), lambda i: (i, 0)),
            pl.BlockSpec((wi.shape[0], wf), lambda i: (0, 0)),
            pl.BlockSpec((wj.shape[0], wf), lambda i: (0, 0)),
            pl.BlockSpec((1, wf), lambda i: (0, 0)),
        ],
        out_specs=[
            pl.BlockSpec((NB, wf), lambda i: (i, 0)),
            pl.BlockSpec((NB, wf), lambda i: (i, 0)),
        ],
        out_shape=[
            jax.ShapeDtypeStruct((NP, wf), jnp.float32),
            jax.ShapeDtypeStruct((NP, wf), jnp.float32),
        ],
    )(xp, wi, wj, c)


def _post_math(h, u, ssum, ssq, smn, smx, d, wpp, bpost, wlin, blin, wf):
    degc = jnp.maximum(d, 1.0)
    nz = (d > 0.0).astype(jnp.float32)
    sm = ssum / degc
    mean = sm + u * nz
    var = ssq / degc - sm * sm
    std = jnp.sqrt(jnp.maximum(var, 0.0) + 1e-5)
    mn = (smn + u) * nz
    mx = (smx + u) * nz
    logd = jnp.log(degc + 1.0)
    s2 = logd * (1.0 / AVG_DEG_LOG)
    s3 = AVG_DEG_LOG / logd
    o = jnp.dot(h, wpp[0:wf], preferred_element_type=jnp.float32)
    for k, p in enumerate((mean, mn, mx, std)):
        o += jnp.dot(p, wpp[(1 + k) * wf:(2 + k) * wf],
                     preferred_element_type=jnp.float32)
        o += jnp.dot(p * s2, wpp[(5 + k) * wf:(6 + k) * wf],
                     preferred_element_type=jnp.float32)
        o += jnp.dot(p * s3, wpp[(9 + k) * wf:(10 + k) * wf],
                     preferred_element_type=jnp.float32)
    o = o + bpost
    return jnp.dot(o, wlin, preferred_element_type=jnp.float32) + blin


def _post_body(wf, h_ref, u_ref, ssum_ref, ssq_ref, smn_ref, smx_ref, d_ref,
               wpp_ref, bpost_ref, wlin_ref, blin_ref,
               win_ref, cn_ref, wjn_ref,
               h2_ref, an_ref, bn_ref):
    o = _post_math(h_ref[...], u_ref[...], ssum_ref[...], ssq_ref[...],
                   smn_ref[...], smx_ref[...], d_ref[...], wpp_ref[...],
                   bpost_ref[...], wlin_ref[...], blin_ref[...], wf)
    o = jnp.where(o > 0, o, 0.01 * o)
    h2_ref[...] = o
    an_ref[...] = jnp.dot(o, win_ref[...],
                          preferred_element_type=jnp.float32) + cn_ref[...]
    bn_ref[...] = jnp.dot(o, wjn_ref[...],
                          preferred_element_type=jnp.float32)


def _post_pre(h, u, ssum, ssq, smn, smx, deg2, wpp, bpost, wlin, blin,
              win, cn, wjn, wf):
    wfn = win.shape[1]
    return pl.pallas_call(
        functools.partial(_post_body, wf),
        grid=(NGRID,),
        in_specs=[
            pl.BlockSpec((NB, h.shape[1]), lambda i: (i, 0)),
            pl.BlockSpec((NB, wf), lambda i: (i, 0)),
            pl.BlockSpec((NB, wf), lambda i: (i, 0)),
            pl.BlockSpec((NB, wf), lambda i: (i, 0)),
            pl.BlockSpec((NB, wf), lambda i: (i, 0)),
            pl.BlockSpec((NB, wf), lambda i: (i, 0)),
            pl.BlockSpec((NB, 1), lambda i: (i, 0)),
            pl.BlockSpec(wpp.shape, lambda i: (0, 0)),
            pl.BlockSpec((1, HO), lambda i: (0, 0)),
            pl.BlockSpec((HO, HO), lambda i: (0, 0)),
            pl.BlockSpec((1, HO), lambda i: (0, 0)),
            pl.BlockSpec((HO, wfn), lambda i: (0, 0)),
            pl.BlockSpec((1, wfn), lambda i: (0, 0)),
            pl.BlockSpec((HO, wfn), lambda i: (0, 0)),
        ],
        out_specs=[
            pl.BlockSpec((NB, HO), lambda i: (i, 0)),
            pl.BlockSpec((NB, wfn), lambda i: (i, 0)),
            pl.BlockSpec((NB, wfn), lambda i: (i, 0)),
        ],
        out_shape=[
            jax.ShapeDtypeStruct((NP, HO), jnp.float32),
            jax.ShapeDtypeStruct((NP, wfn), jnp.float32),
            jax.ShapeDtypeStruct((NP, wfn), jnp.float32),
        ],
    )(h, u, ssum, ssq, smn, smx, deg2, wpp, bpost, wlin, blin, win, cn, wjn)


def _final_body(wf, h_ref, u_ref, ssum_ref, ssq_ref, smn_ref, smx_ref, d_ref,
                wpp_ref, bpost_ref, wlin_ref, blin_ref, bt_ref,
                w1_ref, b1_ref, w2_ref, b2_ref,
                out_ref, pacc, cacc):
    i = pl.program_id(0)

    @pl.when(i == 0)
    def _():
        pacc[...] = jnp.zeros_like(pacc)
        cacc[...] = jnp.zeros_like(cacc)

    o = _post_math(h_ref[...], u_ref[...], ssum_ref[...], ssq_ref[...],
                   smn_ref[...], smx_ref[...], d_ref[...], wpp_ref[...],
                   bpost_ref[...], wlin_ref[...], blin_ref[...], wf)
    bt = bt_ref[...]
    oh = (bt == lax.broadcasted_iota(jnp.int32, (NB, NG), 1)).astype(
        jnp.float32)
    pacc[...] += lax.dot_general(oh, o, (((0,), (0,)), ((), ())),
                                 preferred_element_type=jnp.float32)
    cacc[...] += lax.dot_general(
        oh, jnp.ones((NB, 8), jnp.float32), (((0,), (0,)), ((), ())),
        preferred_element_type=jnp.float32)

    @pl.when(i == NGRID - 1)
    def _():
        cnt = jnp.maximum(cacc[...][:, 0:1], 1.0)
        pooled = pacc[...] / cnt
        z = jnp.dot(pooled, w1_ref[...],
                    preferred_element_type=jnp.float32) + b1_ref[...]
        z = jnp.maximum(z, 0.0)
        out_ref[...] = jnp.dot(z, w2_ref[...],
                               preferred_element_type=jnp.float32) + b2_ref[...]


def _final(h, u, ssum, ssq, smn, smx, deg2, wpp, bpost, wlin, blin, batch2,
           w1, b1, w2p, b2p, wf):
    return pl.pallas_call(
        functools.partial(_final_body, wf),
        grid=(NGRID,),
        in_specs=[
            pl.BlockSpec((NB, h.shape[1]), lambda i: (i, 0)),
            pl.BlockSpec((NB, wf), lambda i: (i, 0)),
            pl.BlockSpec((NB, wf), lambda i: (i, 0)),
            pl.BlockSpec((NB, wf), lambda i: (i, 0)),
            pl.BlockSpec((NB, wf), lambda i: (i, 0)),
            pl.BlockSpec((NB, wf), lambda i: (i, 0)),
            pl.BlockSpec((NB, 1), lambda i: (i, 0)),
            pl.BlockSpec(wpp.shape, lambda i: (0, 0)),
            pl.BlockSpec((1, HO), lambda i: (0, 0)),
            pl.BlockSpec((HO, HO), lambda i: (0, 0)),
            pl.BlockSpec((1, HO), lambda i: (0, 0)),
            pl.BlockSpec((NB, 1), lambda i: (i, 0)),
            pl.BlockSpec((HO, 32), lambda i: (0, 0)),
            pl.BlockSpec((1, 32), lambda i: (0, 0)),
            pl.BlockSpec((32, 128), lambda i: (0, 0)),
            pl.BlockSpec((1, 128), lambda i: (0, 0)),
        ],
        out_specs=[pl.BlockSpec((NG, 128), lambda i: (0, 0))],
        out_shape=[jax.ShapeDtypeStruct((NG, 128), jnp.float32)],
        scratch_shapes=[
            pltpu.VMEM((NG, HO), jnp.float32),
            pltpu.VMEM((NG, 8), jnp.float32),
        ],
    )(h, u, ssum, ssq, smn, smx, deg2, wpp, bpost, wlin, blin, batch2,
      w1, b1, w2p, b2p)[0]


def _prep_conv(p, f_real, wf):
    wpre = p["Wpre"]
    wi = wpre[:f_real]
    wj = wpre[f_real:2 * f_real]
    we2 = wpre[2 * f_real:3 * f_real]
    w = p["We"][0] @ we2
    c = p["be"] @ we2 + p["bpre"]
    pad = wf - f_real
    wi = jnp.pad(wi, ((0, pad), (0, pad)))
    wj = jnp.pad(wj, ((0, pad), (0, pad)))
    w = jnp.pad(w, (0, pad))
    c = jnp.pad(c, (0, pad))
    wpost = p["Wpost"]
    blocks = [jnp.pad(wpost[k * f_real:(k + 1) * f_real], ((0, pad), (0, 0)))
              for k in range(13)]
    wpp = jnp.concatenate(blocks, axis=0)
    return (wi, wj, c[None, :], w, wpp, p["bpost"][None, :],
            p["Wlin"], p["blin"][None, :])


def kernel(x, edge_index, edge_attr, batch, conv1, conv2, conv3, lin):
    src = edge_index[0].astype(jnp.int32)
    dst = edge_index[1].astype(jnp.int32)
    ea = edge_attr[:, 0]

    wi1, wj1, c1, w1v, wpp1, bp1, wl1, bl1 = _prep_conv(conv1, 7, 16)
    wi2, wj2, c2, w2v, wpp2, bp2, wl2, bl2 = _prep_conv(conv2, 64, 64)
    wi3, wj3, c3, w3v, wpp3, bp3, wl3, bl3 = _prep_conv(conv3, 64, 64)

    xp = jnp.pad(x, ((0, NP - N), (0, 16 - 7)))
    batch2 = jnp.pad(batch.astype(jnp.int32), (0, NP - N),
                     constant_values=NG)[:, None]
    w2p = jnp.pad(lin["W2"], ((0, 0), (0, 128 - NCLS)))
    b2p = jnp.pad(lin["b2"], (0, 128 - NCLS))[None, :]

    pidx, eas, choff, deg = _bucket(dst, src, ea)
    deg2 = deg[:, None]

    a1, b1 = _pre1(xp, wi1, wj1, c1)
    s1, q1, mn1, mx1 = _edge_stats(b1, pidx, eas, choff, w1v, 16)
    h2, a2, b2 = _post_pre(xp, a1, s1, q1, mn1, mx1, deg2, wpp1, bp1, wl1,
                           bl1, wi2, c2, wj2, 16)
    s2, q2, mn2, mx2 = _edge_stats(b2, pidx, eas, choff, w2v, 64)
    h3, a3, b3 = _post_pre(h2, a2, s2, q2, mn2, mx2, deg2, wpp2, bp2, wl2,
                           bl2, wi3, c3, wj3, 64)
    s3, q3, mn3, mx3 = _edge_stats(b3, pidx, eas, choff, w3v, 64)
    out = _final(h3, a3, s3, q3, mn3, mx3, deg2, wpp3, bp3, wl3, bl3, batch2,
                 lin["W1"], lin["b1"][None, :], w2p, b2p, 64)
    return out[:, :NCLS]


# speedup vs baseline: 4.6724x; 1.0077x over previous
import functools
import math

import jax
import jax.numpy as jnp
from jax import lax
from jax.experimental import pallas as pl
from jax.experimental.pallas import tpu as pltpu
from jax.experimental.pallas import tpu_sc as plsc

N = 50000
E = 800000
NG = 128
NCLS = 5
HO = 64

NC = 2
NS = 16
NW = NC * NS
RANGE = 1664
NP = NW * RANGE
CH = 128
NCHUNK = RANGE // CH
CAP = 28672
BA = 6400
BG = 512

AVG_DEG_LOG = math.log(17.0)
FMAX = 3.0e38

_SC_PARAMS = pltpu.CompilerParams(needs_layout_passes=False,
                                  use_tc_tiling_on_sc=False)


@functools.cache
def _mesh():
    return plsc.VectorSubcoreMesh(core_axis_name="c", subcore_axis_name="s")


def _wid():
    return lax.axis_index("s") * NC + lax.axis_index("c")


def _splat(v):
    return jnp.full((16,), v, jnp.int32)


def _sget(ref, i):
    return plsc.load_gather(ref, indices=[_splat(i)])[0]


def _sset(ref, i, v):
    plsc.store_scatter(ref, [_splat(i)], jnp.full((16,), v))


def _bucket_body(dst_h, src_h, ea_h, pidx_h, eas_h, choff_h, deg_h,
                 dbuf, sbuf, ebuf, stg_p, stg_e, hist16, off16, deg16, pos2,
                 degf, chof, sem):
    wid = _wid()
    lo = wid * RANGE
    lanes = lax.iota(jnp.int32, 16)
    ones16 = jnp.ones((16,), jnp.int32)

    def z16(v, _):
        stg_p[pl.ds(v * 16, 16)] = jnp.zeros((16,), jnp.int32)
        return 0
    lax.fori_loop(0, CAP // 16, z16, 0)

    def zh(v, _):
        hist16[pl.ds(v * 16, 16)] = jnp.zeros((16,), jnp.int32)
        return 0
    lax.fori_loop(0, 224 // 16, zh, 0)

    def zp(v, _):
        pos2[v >> 3, pl.ds((v & 7) * 16, 16)] = (
            lax.iota(jnp.int32, 16) + v * 16 + 128)
        return 0
    lax.fori_loop(0, CAP // 16, zp, 0)

    def blk(b, cur):
        base = b * BA
        pltpu.sync_copy(dst_h.at[pl.ds(base, BA)], dbuf)
        pltpu.sync_copy(src_h.at[pl.ds(base, BA)], sbuf)
        pltpu.sync_copy(ea_h.at[pl.ds(base, BA)], ebuf)

        def vec(i, cur):
            for k in range(4):
                o = i * 64 + k * 16
                d = dbuf[pl.ds(o, 16)]
                s = sbuf[pl.ds(o, 16)]
                a = ebuf[pl.ds(o, 16)]
                msk = (d >= lo) & (d < lo + RANGE)
                pk = ((d - lo) << 16) | s
                plsc.store_compressed(stg_p.at[pl.ds(cur, 16)], pk, mask=msk)
                plsc.store_compressed(stg_e.at[pl.ds(cur, 16)], a, mask=msk)
                cur = cur + plsc.all_reduce_population_count(msk)[0]
            return cur
        return lax.fori_loop(0, BA // 64, vec, cur)
    cnt = lax.fori_loop(0, E // BA, blk, jnp.int32(0))

    nfull = cnt >> 4
    tail = cnt & 15
    tm = lanes < tail

    def hv(v, _):
        pk = stg_p[pl.ds(v * 16, 16)]
        plsc.addupdate_scatter(hist16, [(pk >> 23) * 16 + lanes], ones16)
        return 0
    lax.fori_loop(0, nfull, hv, 0)
    pkt = stg_p[pl.ds(nfull * 16, 16)]
    plsc.addupdate_scatter(hist16, [(pkt >> 23) * 16 + lanes], ones16,
                           mask=tm)

    def pre(cc, run):
        start = (run + 7) & ~7
        _sset(chof, cc, start)
        sl = pl.ds(cc * 16, 16)
        hv16 = hist16[sl]
        off16[sl] = plsc.cumsum(hv16) - hv16 + start
        tot = jnp.sum(hv16)
        _sset(chof, 16 + cc, tot)
        return start + tot
    lax.fori_loop(0, NCHUNK, pre, jnp.int32(0))
    _sset(chof, 15, cnt)
    pltpu.sync_copy(chof, choff_h.at[wid])

    def qv(v, _):
        pk = stg_p[pl.ds(v * 16, 16)]
        idxv = (pk >> 23) * 16 + lanes
        p = plsc.load_gather(off16, [idxv])
        plsc.store_scatter(off16, [idxv], p + 1)
        pos2[v >> 3, pl.ds((v & 7) * 16, 16)] = p
        return 0
    lax.fori_loop(0, nfull, qv, 0)
    idxt = (pkt >> 23) * 16 + lanes
    pt = plsc.load_gather(off16, [idxt], mask=tm)
    plsc.store_scatter(off16, [idxt], pt + 1, mask=tm)
    tsl = pl.ds((nfull * 16) & 127, 16)
    pos2[nfull >> 3, tsl] = jnp.where(tm, pt, pos2[nfull >> 3, tsl])

    nrows = (cnt + 127) >> 7

    def _srow_descs(j):
        return (pltpu.make_async_copy(stg_p.at[pl.ds(j * 128, 128)],
                                      pidx_h.at[wid].at[pos2.at[j]], sem),
                pltpu.make_async_copy(stg_e.at[pl.ds(j * 128, 128)],
                                      eas_h.at[wid].at[pos2.at[j]], sem))

    def sgrp(g, _):
        for k in range(8):
            @pl.when(g * 8 + k < nrows)
            def _():
                d1, d2 = _srow_descs(g * 8 + k)
                d1.start()
                d2.start()
        for k in range(8):
            @pl.when(g * 8 + k < nrows)
            def _():
                d1, d2 = _srow_descs(g * 8 + k)
                d1.wait()
                d2.wait()
        return 0
    lax.fori_loop(0, (nrows + 7) >> 3, sgrp, 0)

    def dchunk(c, _):
        def zd(v, _):
            deg16[pl.ds(v * 16, 16)] = jnp.zeros((16,), jnp.int32)
            return 0
        lax.fori_loop(0, CH, zd, 0)

        def sv(v, _):
            pk = stg_p[pl.ds(v * 16, 16)]
            msk = ((pk >> 23) == c) & ((v * 16 + lanes) < cnt)
            idxv = ((pk >> 16) - c * CH) * 16 + lanes
            idxv = jnp.clip(idxv, 0, CH * 16 - 1)
            plsc.addupdate_scatter(deg16, [idxv], ones16, mask=msk)
            return 0
        lax.fori_loop(0, (cnt + 15) >> 4, sv, 0)

        def rd(n, _):
            s = jnp.sum(deg16[pl.ds(n * 16, 16)])
            _sset(degf, c * CH + n, s.astype(jnp.float32))
            return 0
        lax.fori_loop(0, CH, rd, 0)
        return 0
    lax.fori_loop(0, NCHUNK, dchunk, 0)
    pltpu.sync_copy(degf, deg_h.at[pl.ds(lo, RANGE)])


def _bucket(dst, src, ea):
    f = pl.kernel(
        _bucket_body,
        out_type=(
            jax.ShapeDtypeStruct((NW, CAP), jnp.int32),
            jax.ShapeDtypeStruct((NW, CAP), jnp.float32),
            jax.ShapeDtypeStruct((NW, 32), jnp.int32),
            jax.ShapeDtypeStruct((NP,), jnp.float32),
        ),
        mesh=_mesh(),
        compiler_params=_SC_PARAMS,
        scratch_types=[
            pltpu.VMEM((BA,), jnp.int32),
            pltpu.VMEM((BA,), jnp.int32),
            pltpu.VMEM((BA,), jnp.float32),
            pltpu.VMEM((CAP + 16,), jnp.int32),
            pltpu.VMEM((CAP + 16,), jnp.float32),
            pltpu.VMEM((224,), jnp.int32),
            pltpu.VMEM((224,), jnp.int32),
            pltpu.VMEM((CH * 16,), jnp.int32),
            pltpu.VMEM((CAP // 128, 128), jnp.int32),
            pltpu.VMEM((RANGE,), jnp.float32),
            pltpu.VMEM((32,), jnp.int32),
            pltpu.SemaphoreType.DMA,
        ],
    )
    return f(dst, src, ea)


def _edge_body(wf, bp_h, pidx_h, eas_h, choff_h, w_h,
               ssum_h, ssq_h, smn_h, smx_h,
               pb0, pb1, eb0, eb1, ix0, ix1, rw0, rw1,
               asum, asq, amn, amx, wv, chof,
               smi0, smi1, smg0, smg1):
    wid = _wid()
    nb = wid * RANGE
    pltpu.sync_copy(choff_h.at[wid], chof)
    pltpu.sync_copy(w_h, wv)
    nj = wf // 16
    pbufs = (pb0, pb1)
    ebufs = (eb0, eb1)
    ixs = (ix0, ix1)
    rws = (rw0, rw1)
    smis = (smi0, smi1)
    smgs = (smg0, smg1)

    def chunk(c, _):
        e0 = _sget(chof, c)
        ec = _sget(chof, 16 + c)
        nbase = nb + c * CH

        def init_v(v, _):
            sl = pl.ds(v * 16, 16)
            asum[sl] = jnp.zeros((16,), jnp.float32)
            asq[sl] = jnp.zeros((16,), jnp.float32)
            amn[sl] = jnp.full((16,), FMAX, jnp.float32)
            amx[sl] = jnp.full((16,), -FMAX, jnp.float32)
            return 0
        lax.fori_loop(0, CH * wf // 16, init_v, 0)

        nblk = (ec + BG - 1) >> 9

        def in_descs(b, par):
            base = pl.multiple_of(e0 + b * BG, 8)
            d1 = pltpu.make_async_copy(pidx_h.at[wid].at[pl.ds(base, BG)],
                                       pbufs[par].at[pl.ds(0, BG)], smis[par])
            d2 = pltpu.make_async_copy(eas_h.at[wid].at[pl.ds(base, BG)],
                                       ebufs[par].at[pl.ds(0, BG)], smis[par])
            return d1, d2

        def g_descs(par):
            return [pltpu.make_async_copy(bp_h.at[ixs[par].at[j]],
                                          rws[par].at[pl.ds(j * 128, 128)],
                                          smgs[par])
                    for j in range(BG // 128)]

        def fire_in(b, par):
            d1, d2 = in_descs(b, par)
            d1.start()
            d2.start()

        def wait_in(b, par):
            d1, d2 = in_descs(b, par)
            d1.wait()
            d2.wait()

        def idx_and_gather(par):
            def vi(v, _):
                s = pbufs[par][pl.ds(v * 16, 16)] & 0xFFFF
                ixs[par][v >> 3, pl.ds((v & 7) * 16, 16)] = jnp.minimum(
                    s, jnp.int32(N - 1))
                return 0
            lax.fori_loop(0, BG // 16, vi, 0)
            for d in g_descs(par):
                d.start()

        def edges(b, par):
            nrem = jnp.minimum(jnp.int32(BG), ec - b * BG)
            pbuf, ebuf, rows = pbufs[par], ebufs[par], rws[par]
            cbase = c * (CH * wf)

            def one(e):
                pk = pbuf[pl.ds(e, 16)][0]
                off = (pk >> 16) * wf - cbase
                a = ebuf[pl.ds(e, 16)][0]
                for j in range(nj):
                    sl = pl.ds(off + j * 16, 16)
                    t = rows[e, pl.ds(j * 16, 16)] + a * wv[pl.ds(j * 16, 16)]
                    plsc.addupdate(asum.at[sl], t)
                    plsc.addupdate(asq.at[sl], t * t)
                    amn[sl] = jnp.minimum(amn[sl], t)
                    amx[sl] = jnp.maximum(amx[sl], t)

            def epair(q, _):
                one(2 * q)

                @pl.when(2 * q + 1 < nrem)
                def _():
                    one(2 * q + 1)
                return 0
            lax.fori_loop(0, (nrem + 1) >> 1, epair, 0)

        @pl.when(nblk > 0)
        def _():
            fire_in(0, 0)
            wait_in(0, 0)
            idx_and_gather(0)

        @pl.when(nblk > 1)
        def _():
            fire_in(1, 1)

        def pair(q, _):
            for par in (0, 1):
                b = 2 * q + par

                @pl.when(b < nblk)
                def _():
                    @pl.when(b + 1 < nblk)
                    def _():
                        wait_in(b + 1, 1 - par)
                        idx_and_gather(1 - par)
                    for d in g_descs(par):
                        d.wait()
                    edges(b, par)

                    @pl.when(b + 2 < nblk)
                    def _():
                        fire_in(b + 2, par)
            return 0
        lax.fori_loop(0, (nblk + 1) >> 1, pair, 0)

        fb = nbase * wf
        pltpu.sync_copy(asum, ssum_h.at[pl.ds(fb, CH * wf)])
        pltpu.sync_copy(asq, ssq_h.at[pl.ds(fb, CH * wf)])
        pltpu.sync_copy(amn, smn_h.at[pl.ds(fb, CH * wf)])
        pltpu.sync_copy(amx, smx_h.at[pl.ds(fb, CH * wf)])
        return 0
    lax.fori_loop(0, NCHUNK, chunk, 0)


def _edge_stats(bp, pidx, eas, choff, w, wf):
    f = pl.kernel(
        functools.partial(_edge_body, wf),
        out_type=(
            jax.ShapeDtypeStruct((NP * wf,), jnp.float32),
            jax.ShapeDtypeStruct((NP * wf,), jnp.float32),
            jax.ShapeDtypeStruct((NP * wf,), jnp.float32),
            jax.ShapeDtypeStruct((NP * wf,), jnp.float32),
        ),
        mesh=_mesh(),
        compiler_params=_SC_PARAMS,
        scratch_types=[
            pltpu.VMEM((BG + 16,), jnp.int32),
            pltpu.VMEM((BG + 16,), jnp.int32),
            pltpu.VMEM((BG + 16,), jnp.float32),
            pltpu.VMEM((BG + 16,), jnp.float32),
            pltpu.VMEM((BG // 128, 128), jnp.int32),
            pltpu.VMEM((BG // 128, 128), jnp.int32),
            pltpu.VMEM((BG, wf), jnp.float32),
            pltpu.VMEM((BG, wf), jnp.float32),
            pltpu.VMEM((CH * wf,), jnp.float32),
            pltpu.VMEM((CH * wf,), jnp.float32),
            pltpu.VMEM((CH * wf,), jnp.float32),
            pltpu.VMEM((CH * wf,), jnp.float32),
            pltpu.VMEM((wf,), jnp.float32),
            pltpu.VMEM((32,), jnp.int32),
            pltpu.SemaphoreType.DMA,
            pltpu.SemaphoreType.DMA,
            pltpu.SemaphoreType.DMA,
            pltpu.SemaphoreType.DMA,
        ],
    )
    s, q, mn, mx = f(bp, pidx, eas, choff, w)
    rs = lambda t: t.reshape(NP, wf)
    return rs(s), rs(q), rs(mn), rs(mx)


NB = 256
NGRID = NP // NB


def _pre1_body(x_ref, wi_ref, wj_ref, c_ref, a_ref, b_ref):
    x = x_ref[...]
    a_ref[...] = jnp.dot(x, wi_ref[...],
                         preferred_element_type=jnp.float32) + c_ref[...]
    b_ref[...] = jnp.dot(x, wj_ref[...], preferred_element_type=jnp.float32)


def _pre1(xp, wi, wj, c):
    wf = wi.shape[1]
    return pl.pallas_call(
        _pre1_body,
        grid=(NGRID,),
        in_specs=[
            pl.BlockSpec((NB, xp.shape[1]), lambda i: (i, 0)),
            pl.BlockSpec((wi.shape[0], wf), lambda i: (0, 0)),
            pl.BlockSpec((wj.shape[0], wf), lambda i: (0, 0)),
            pl.BlockSpec((1, wf), lambda i: (0, 0)),
        ],
        out_specs=[
            pl.BlockSpec((NB, wf), lambda i: (i, 0)),
            pl.BlockSpec((NB, wf), lambda i: (i, 0)),
        ],
        out_shape=[
            jax.ShapeDtypeStruct((NP, wf), jnp.float32),
            jax.ShapeDtypeStruct((NP, wf), jnp.float32),
        ],
    )(xp, wi, wj, c)


def _post_math(h, u, ssum, ssq, smn, smx, d, wpp, bpost, wlin, blin, wf):
    degc = jnp.maximum(d, 1.0)
    nz = (d > 0.0).astype(jnp.float32)
    sm = ssum / degc
    mean = sm + u * nz
    var = ssq / degc - sm * sm
    std = jnp.sqrt(jnp.maximum(var, 0.0) + 1e-5)
    mn = (smn + u) * nz
    mx = (smx + u) * nz
    logd = jnp.log(degc + 1.0)
    s2 = logd * (1.0 / AVG_DEG_LOG)
    s3 = AVG_DEG_LOG / logd
    o = jnp.dot(h, wpp[0:wf], preferred_element_type=jnp.float32)
    for k, p in enumerate((mean, mn, mx, std)):
        o += jnp.dot(p, wpp[(1 + k) * wf:(2 + k) * wf],
                     preferred_element_type=jnp.float32)
        o += jnp.dot(p * s2, wpp[(5 + k) * wf:(6 + k) * wf],
                     preferred_element_type=jnp.float32)
        o += jnp.dot(p * s3, wpp[(9 + k) * wf:(10 + k) * wf],
                     preferred_element_type=jnp.float32)
    o = o + bpost
    return jnp.dot(o, wlin, preferred_element_type=jnp.float32) + blin


def _post_body(wf, h_ref, u_ref, ssum_ref, ssq_ref, smn_ref, smx_ref, d_ref,
               wpp_ref, bpost_ref, wlin_ref, blin_ref,
               win_ref, cn_ref, wjn_ref,
               h2_ref, an_ref, bn_ref):
    o = _post_math(h_ref[...], u_ref[...], ssum_ref[...], ssq_ref[...],
                   smn_ref[...], smx_ref[...], d_ref[...], wpp_ref[...],
                   bpost_ref[...], wlin_ref[...], blin_ref[...], wf)
    o = jnp.where(o > 0, o, 0.01 * o)
    h2_ref[...] = o
    an_ref[...] = jnp.dot(o, win_ref[...],
                          preferred_element_type=jnp.float32) + cn_ref[...]
    bn_ref[...] = jnp.dot(o, wjn_ref[...],
                          preferred_element_type=jnp.float32)


def _post_pre(h, u, ssum, ssq, smn, smx, deg2, wpp, bpost, wlin, blin,
              win, cn, wjn, wf):
    wfn = win.shape[1]
    return pl.pallas_call(
        functools.partial(_post_body, wf),
        grid=(NGRID,),
        in_specs=[
            pl.BlockSpec((NB, h.shape[1]), lambda i: (i, 0)),
            pl.BlockSpec((NB, wf), lambda i: (i, 0)),
            pl.BlockSpec((NB, wf), lambda i: (i, 0)),
            pl.BlockSpec((NB, wf), lambda i: (i, 0)),
            pl.BlockSpec((NB, wf), lambda i: (i, 0)),
            pl.BlockSpec((NB, wf), lambda i: (i, 0)),
            pl.BlockSpec((NB, 1), lambda i: (i, 0)),
            pl.BlockSpec(wpp.shape, lambda i: (0, 0)),
            pl.BlockSpec((1, HO), lambda i: (0, 0)),
            pl.BlockSpec((HO, HO), lambda i: (0, 0)),
            pl.BlockSpec((1, HO), lambda i: (0, 0)),
            pl.BlockSpec((HO, wfn), lambda i: (0, 0)),
            pl.BlockSpec((1, wfn), lambda i: (0, 0)),
            pl.BlockSpec((HO, wfn), lambda i: (0, 0)),
        ],
        out_specs=[
            pl.BlockSpec((NB, HO), lambda i: (i, 0)),
            pl.BlockSpec((NB, wfn), lambda i: (i, 0)),
            pl.BlockSpec((NB, wfn), lambda i: (i, 0)),
        ],
        out_shape=[
            jax.ShapeDtypeStruct((NP, HO), jnp.float32),
            jax.ShapeDtypeStruct((NP, wfn), jnp.float32),
            jax.ShapeDtypeStruct((NP, wfn), jnp.float32),
        ],
    )(h, u, ssum, ssq, smn, smx, deg2, wpp, bpost, wlin, blin, win, cn, wjn)


def _final_body(wf, h_ref, u_ref, ssum_ref, ssq_ref, smn_ref, smx_ref, d_ref,
                wpp_ref, bpost_ref, wlin_ref, blin_ref, bt_ref,
                w1_ref, b1_ref, w2_ref, b2_ref,
                out_ref, pacc, cacc):
    i = pl.program_id(0)

    @pl.when(i == 0)
    def _():
        pacc[...] = jnp.zeros_like(pacc)
        cacc[...] = jnp.zeros_like(cacc)

    o = _post_math(h_ref[...], u_ref[...], ssum_ref[...], ssq_ref[...],
                   smn_ref[...], smx_ref[...], d_ref[...], wpp_ref[...],
                   bpost_ref[...], wlin_ref[...], blin_ref[...], wf)
    bt = bt_ref[...]
    oh = (bt == lax.broadcasted_iota(jnp.int32, (NB, NG), 1)).astype(
        jnp.float32)
    pacc[...] += lax.dot_general(oh, o, (((0,), (0,)), ((), ())),
                                 preferred_element_type=jnp.float32)
    cacc[...] += lax.dot_general(
        oh, jnp.ones((NB, 8), jnp.float32), (((0,), (0,)), ((), ())),
        preferred_element_type=jnp.float32)

    @pl.when(i == NGRID - 1)
    def _():
        cnt = jnp.maximum(cacc[...][:, 0:1], 1.0)
        pooled = pacc[...] / cnt
        z = jnp.dot(pooled, w1_ref[...],
                    preferred_element_type=jnp.float32) + b1_ref[...]
        z = jnp.maximum(z, 0.0)
        out_ref[...] = jnp.dot(z, w2_ref[...],
                               preferred_element_type=jnp.float32) + b2_ref[...]


def _final(h, u, ssum, ssq, smn, smx, deg2, wpp, bpost, wlin, blin, batch2,
           w1, b1, w2p, b2p, wf):
    return pl.pallas_call(
        functools.partial(_final_body, wf),
        grid=(NGRID,),
        in_specs=[
            pl.BlockSpec((NB, h.shape[1]), lambda i: (i, 0)),
            pl.BlockSpec((NB, wf), lambda i: (i, 0)),
            pl.BlockSpec((NB, wf), lambda i: (i, 0)),
            pl.BlockSpec((NB, wf), lambda i: (i, 0)),
            pl.BlockSpec((NB, wf), lambda i: (i, 0)),
            pl.BlockSpec((NB, wf), lambda i: (i, 0)),
            pl.BlockSpec((NB, 1), lambda i: (i, 0)),
            pl.BlockSpec(wpp.shape, lambda i: (0, 0)),
            pl.BlockSpec((1, HO), lambda i: (0, 0)),
            pl.BlockSpec((HO, HO), lambda i: (0, 0)),
            pl.BlockSpec((1, HO), lambda i: (0, 0)),
            pl.BlockSpec((NB, 1), lambda i: (i, 0)),
            pl.BlockSpec((HO, 32), lambda i: (0, 0)),
            pl.BlockSpec((1, 32), lambda i: (0, 0)),
            pl.BlockSpec((32, 128), lambda i: (0, 0)),
            pl.BlockSpec((1, 128), lambda i: (0, 0)),
        ],
        out_specs=[pl.BlockSpec((NG, 128), lambda i: (0, 0))],
        out_shape=[jax.ShapeDtypeStruct((NG, 128), jnp.float32)],
        scratch_shapes=[
            pltpu.VMEM((NG, HO), jnp.float32),
            pltpu.VMEM((NG, 8), jnp.float32),
        ],
    )(h, u, ssum, ssq, smn, smx, deg2, wpp, bpost, wlin, blin, batch2,
      w1, b1, w2p, b2p)[0]


def _prep_conv(p, f_real, wf):
    wpre = p["Wpre"]
    wi = wpre[:f_real]
    wj = wpre[f_real:2 * f_real]
    we2 = wpre[2 * f_real:3 * f_real]
    w = p["We"][0] @ we2
    c = p["be"] @ we2 + p["bpre"]
    pad = wf - f_real
    wi = jnp.pad(wi, ((0, pad), (0, pad)))
    wj = jnp.pad(wj, ((0, pad), (0, pad)))
    w = jnp.pad(w, (0, pad))
    c = jnp.pad(c, (0, pad))
    wpost = p["Wpost"]
    blocks = [jnp.pad(wpost[k * f_real:(k + 1) * f_real], ((0, pad), (0, 0)))
              for k in range(13)]
    wpp = jnp.concatenate(blocks, axis=0)
    return (wi, wj, c[None, :], w, wpp, p["bpost"][None, :],
            p["Wlin"], p["blin"][None, :])


def kernel(x, edge_index, edge_attr, batch, conv1, conv2, conv3, lin):
    src = edge_index[0].astype(jnp.int32)
    dst = edge_index[1].astype(jnp.int32)
    ea = edge_attr[:, 0]

    wi1, wj1, c1, w1v, wpp1, bp1, wl1, bl1 = _prep_conv(conv1, 7, 16)
    wi2, wj2, c2, w2v, wpp2, bp2, wl2, bl2 = _prep_conv(conv2, 64, 64)
    wi3, wj3, c3, w3v, wpp3, bp3, wl3, bl3 = _prep_conv(conv3, 64, 64)

    xp = jnp.pad(x, ((0, NP - N), (0, 16 - 7)))
    batch2 = jnp.pad(batch.astype(jnp.int32), (0, NP - N),
                     constant_values=NG)[:, None]
    w2p = jnp.pad(lin["W2"], ((0, 0), (0, 128 - NCLS)))
    b2p = jnp.pad(lin["b2"], (0, 128 - NCLS))[None, :]

    pidx, eas, choff, deg = _bucket(dst, src, ea)
    deg2 = deg[:, None]

    a1, b1 = _pre1(xp, wi1, wj1, c1)
    s1, q1, mn1, mx1 = _edge_stats(b1, pidx, eas, choff, w1v, 16)
    h2, a2, b2 = _post_pre(xp, a1, s1, q1, mn1, mx1, deg2, wpp1, bp1, wl1,
                           bl1, wi2, c2, wj2, 16)
    s2, q2, mn2, mx2 = _edge_stats(b2, pidx, eas, choff, w2v, 64)
    h3, a3, b3 = _post_pre(h2, a2, s2, q2, mn2, mx2, deg2, wpp2, bp2, wl2,
                           bl2, wi3, c3, wj3, 64)
    s3, q3, mn3, mx3 = _edge_stats(b3, pidx, eas, choff, w3v, 64)
    out = _final(h3, a3, s3, q3, mn3, mx3, deg2, wpp3, bp3, wl3, bl3, batch2,
                 lin["W1"], lin["b1"][None, :], w2p, b2p, 64)
    return out[:, :NCLS]


# speedup vs baseline: 4.6775x; 1.0011x over previous
import functools
import math

import jax
import jax.numpy as jnp
from jax import lax
from jax.experimental import pallas as pl
from jax.experimental.pallas import tpu as pltpu
from jax.experimental.pallas import tpu_sc as plsc

N = 50000
E = 800000
NG = 128
NCLS = 5
HO = 64

NC = 2
NS = 16
NW = NC * NS
RANGE = 1664
NP = NW * RANGE
CH = 128
NCHUNK = RANGE // CH
CAP = 28672
BA = 6400
BG = 512

AVG_DEG_LOG = math.log(17.0)
FMAX = 3.0e38

_SC_PARAMS = pltpu.CompilerParams(needs_layout_passes=False,
                                  use_tc_tiling_on_sc=False)


@functools.cache
def _mesh():
    return plsc.VectorSubcoreMesh(core_axis_name="c", subcore_axis_name="s")


def _wid():
    return lax.axis_index("s") * NC + lax.axis_index("c")


def _splat(v):
    return jnp.full((16,), v, jnp.int32)


def _sget(ref, i):
    return plsc.load_gather(ref, indices=[_splat(i)])[0]


def _sset(ref, i, v):
    plsc.store_scatter(ref, [_splat(i)], jnp.full((16,), v))


def _bucket_body(dst_h, src_h, ea_h, pidx_h, eas_h, choff_h, deg_h,
                 dbuf, sbuf, ebuf, stg_p, stg_e, hist16, off16, deg16, pos2,
                 degf, chof, sem):
    wid = _wid()
    lo = wid * RANGE
    lanes = lax.iota(jnp.int32, 16)
    ones16 = jnp.ones((16,), jnp.int32)

    def z16(v, _):
        stg_p[pl.ds(v * 16, 16)] = jnp.zeros((16,), jnp.int32)
        return 0
    lax.fori_loop(0, CAP // 16, z16, 0)

    def zh(v, _):
        hist16[pl.ds(v * 16, 16)] = jnp.zeros((16,), jnp.int32)
        return 0
    lax.fori_loop(0, 224 // 16, zh, 0)

    def zp(v, _):
        pos2[v >> 3, pl.ds((v & 7) * 16, 16)] = (
            lax.iota(jnp.int32, 16) + v * 16 + 128)
        return 0
    lax.fori_loop(0, CAP // 16, zp, 0)

    def blk(b, cur):
        nblks = E // BA
        bb = b + wid * (nblks // NW)
        bb = jnp.where(bb >= nblks, bb - nblks, bb)
        base = pl.multiple_of(bb * BA, 8)
        pltpu.sync_copy(dst_h.at[pl.ds(base, BA)], dbuf)
        pltpu.sync_copy(src_h.at[pl.ds(base, BA)], sbuf)
        pltpu.sync_copy(ea_h.at[pl.ds(base, BA)], ebuf)

        def vec(i, cur):
            for k in range(4):
                o = i * 64 + k * 16
                d = dbuf[pl.ds(o, 16)]
                s = sbuf[pl.ds(o, 16)]
                a = ebuf[pl.ds(o, 16)]
                msk = (d >= lo) & (d < lo + RANGE)
                pk = ((d - lo) << 16) | s
                plsc.store_compressed(stg_p.at[pl.ds(cur, 16)], pk, mask=msk)
                plsc.store_compressed(stg_e.at[pl.ds(cur, 16)], a, mask=msk)
                cur = cur + plsc.all_reduce_population_count(msk)[0]
            return cur
        return lax.fori_loop(0, BA // 64, vec, cur)
    with jax.named_scope("bkt_filter"):
        cnt = lax.fori_loop(0, E // BA, blk, jnp.int32(0))

    nfull = cnt >> 4
    tail = cnt & 15
    tm = lanes < tail

    def hv(v, _):
        pk = stg_p[pl.ds(v * 16, 16)]
        plsc.addupdate_scatter(hist16, [(pk >> 23) * 16 + lanes], ones16)
        return 0
    with jax.named_scope("bkt_hist"):
        lax.fori_loop(0, nfull, hv, 0)
    pkt = stg_p[pl.ds(nfull * 16, 16)]
    plsc.addupdate_scatter(hist16, [(pkt >> 23) * 16 + lanes], ones16,
                           mask=tm)

    def pre(cc, run):
        start = (run + 7) & ~7
        _sset(chof, cc, start)
        sl = pl.ds(cc * 16, 16)
        hv16 = hist16[sl]
        off16[sl] = plsc.cumsum(hv16) - hv16 + start
        tot = jnp.sum(hv16)
        _sset(chof, 16 + cc, tot)
        return start + tot
    lax.fori_loop(0, NCHUNK, pre, jnp.int32(0))
    _sset(chof, 15, cnt)
    pltpu.sync_copy(chof, choff_h.at[wid])

    def qv(v, _):
        pk = stg_p[pl.ds(v * 16, 16)]
        idxv = (pk >> 23) * 16 + lanes
        p = plsc.load_gather(off16, [idxv])
        plsc.store_scatter(off16, [idxv], p + 1)
        pos2[v >> 3, pl.ds((v & 7) * 16, 16)] = p
        return 0
    with jax.named_scope("bkt_pos"):
        lax.fori_loop(0, nfull, qv, 0)
    idxt = (pkt >> 23) * 16 + lanes
    pt = plsc.load_gather(off16, [idxt], mask=tm)
    plsc.store_scatter(off16, [idxt], pt + 1, mask=tm)
    tsl = pl.ds((nfull * 16) & 127, 16)
    pos2[nfull >> 3, tsl] = jnp.where(tm, pt, pos2[nfull >> 3, tsl])

    nrows = (cnt + 127) >> 7

    def _srow_descs(j):
        return (pltpu.make_async_copy(stg_p.at[pl.ds(j * 128, 128)],
                                      pidx_h.at[wid].at[pos2.at[j]], sem),
                pltpu.make_async_copy(stg_e.at[pl.ds(j * 128, 128)],
                                      eas_h.at[wid].at[pos2.at[j]], sem))

    def sgrp(g, _):
        for k in range(8):
            @pl.when(g * 8 + k < nrows)
            def _():
                d1, d2 = _srow_descs(g * 8 + k)
                d1.start()
                d2.start()
        for k in range(8):
            @pl.when(g * 8 + k < nrows)
            def _():
                d1, d2 = _srow_descs(g * 8 + k)
                d1.wait()
                d2.wait()
        return 0
    with jax.named_scope("bkt_scatter"):
        lax.fori_loop(0, (nrows + 7) >> 3, sgrp, 0)

    def dchunk(c, _):
        def zd(v, _):
            deg16[pl.ds(v * 16, 16)] = jnp.zeros((16,), jnp.int32)
            return 0
        lax.fori_loop(0, CH, zd, 0)

        def sv(v, _):
            pk = stg_p[pl.ds(v * 16, 16)]
            msk = ((pk >> 23) == c) & ((v * 16 + lanes) < cnt)
            idxv = ((pk >> 16) - c * CH) * 16 + lanes
            idxv = jnp.clip(idxv, 0, CH * 16 - 1)
            plsc.addupdate_scatter(deg16, [idxv], ones16, mask=msk)
            return 0
        lax.fori_loop(0, (cnt + 15) >> 4, sv, 0)

        def rd(n, _):
            s = jnp.sum(deg16[pl.ds(n * 16, 16)])
            _sset(degf, c * CH + n, s.astype(jnp.float32))
            return 0
        lax.fori_loop(0, CH, rd, 0)
        return 0
    with jax.named_scope("bkt_deg"):
        lax.fori_loop(0, NCHUNK, dchunk, 0)
    pltpu.sync_copy(degf, deg_h.at[pl.ds(lo, RANGE)])


def _bucket(dst, src, ea):
    f = pl.kernel(
        _bucket_body,
        out_type=(
            jax.ShapeDtypeStruct((NW, CAP), jnp.int32),
            jax.ShapeDtypeStruct((NW, CAP), jnp.float32),
            jax.ShapeDtypeStruct((NW, 32), jnp.int32),
            jax.ShapeDtypeStruct((NP,), jnp.float32),
        ),
        mesh=_mesh(),
        compiler_params=_SC_PARAMS,
        scratch_types=[
            pltpu.VMEM((BA,), jnp.int32),
            pltpu.VMEM((BA,), jnp.int32),
            pltpu.VMEM((BA,), jnp.float32),
            pltpu.VMEM((CAP + 16,), jnp.int32),
            pltpu.VMEM((CAP + 16,), jnp.float32),
            pltpu.VMEM((224,), jnp.int32),
            pltpu.VMEM((224,), jnp.int32),
            pltpu.VMEM((CH * 16,), jnp.int32),
            pltpu.VMEM((CAP // 128, 128), jnp.int32),
            pltpu.VMEM((RANGE,), jnp.float32),
            pltpu.VMEM((32,), jnp.int32),
            pltpu.SemaphoreType.DMA,
        ],
    )
    return f(dst, src, ea)


def _edge_body(wf, bp_h, pidx_h, eas_h, choff_h, w_h,
               ssum_h, ssq_h, smn_h, smx_h,
               pb0, pb1, eb0, eb1, ix0, ix1, rw0, rw1,
               asum, asq, amn, amx, wv, chof,
               smi0, smi1, smg0, smg1):
    wid = _wid()
    nb = wid * RANGE
    pltpu.sync_copy(choff_h.at[wid], chof)
    pltpu.sync_copy(w_h, wv)
    nj = wf // 16
    pbufs = (pb0, pb1)
    ebufs = (eb0, eb1)
    ixs = (ix0, ix1)
    rws = (rw0, rw1)
    smis = (smi0, smi1)
    smgs = (smg0, smg1)

    def chunk(c, _):
        e0 = _sget(chof, c)
        ec = _sget(chof, 16 + c)
        nbase = nb + c * CH

        def init_v(v, _):
            sl = pl.ds(v * 16, 16)
            asum[sl] = jnp.zeros((16,), jnp.float32)
            asq[sl] = jnp.zeros((16,), jnp.float32)
            amn[sl] = jnp.full((16,), FMAX, jnp.float32)
            amx[sl] = jnp.full((16,), -FMAX, jnp.float32)
            return 0
        lax.fori_loop(0, CH * wf // 16, init_v, 0)

        nblk = (ec + BG - 1) >> 9

        def in_descs(b, par):
            base = pl.multiple_of(e0 + b * BG, 8)
            d1 = pltpu.make_async_copy(pidx_h.at[wid].at[pl.ds(base, BG)],
                                       pbufs[par].at[pl.ds(0, BG)], smis[par])
            d2 = pltpu.make_async_copy(eas_h.at[wid].at[pl.ds(base, BG)],
                                       ebufs[par].at[pl.ds(0, BG)], smis[par])
            return d1, d2

        def g_descs(par):
            return [pltpu.make_async_copy(bp_h.at[ixs[par].at[j]],
                                          rws[par].at[pl.ds(j * 128, 128)],
                                          smgs[par])
                    for j in range(BG // 128)]

        def fire_in(b, par):
            d1, d2 = in_descs(b, par)
            d1.start()
            d2.start()

        def wait_in(b, par):
            d1, d2 = in_descs(b, par)
            d1.wait()
            d2.wait()

        def idx_and_gather(par):
            def vi(v, _):
                s = pbufs[par][pl.ds(v * 16, 16)] & 0xFFFF
                ixs[par][v >> 3, pl.ds((v & 7) * 16, 16)] = jnp.minimum(
                    s, jnp.int32(N - 1))
                return 0
            lax.fori_loop(0, BG // 16, vi, 0)
            for d in g_descs(par):
                d.start()

        def edges(b, par):
            nrem = jnp.minimum(jnp.int32(BG), ec - b * BG)
            pbuf, ebuf, rows = pbufs[par], ebufs[par], rws[par]
            cbase = c * (CH * wf)

            def one(e):
                pk = pbuf[pl.ds(e, 16)][0]
                off = (pk >> 16) * wf - cbase
                a = ebuf[pl.ds(e, 16)][0]
                for j in range(nj):
                    sl = pl.ds(off + j * 16, 16)
                    t = rows[e, pl.ds(j * 16, 16)] + a * wv[pl.ds(j * 16, 16)]
                    plsc.addupdate(asum.at[sl], t)
                    plsc.addupdate(asq.at[sl], t * t)
                    amn[sl] = jnp.minimum(amn[sl], t)
                    amx[sl] = jnp.maximum(amx[sl], t)

            def epair(q, _):
                one(2 * q)

                @pl.when(2 * q + 1 < nrem)
                def _():
                    one(2 * q + 1)
                return 0
            lax.fori_loop(0, (nrem + 1) >> 1, epair, 0)

        @pl.when(nblk > 0)
        def _():
            fire_in(0, 0)
            wait_in(0, 0)
            idx_and_gather(0)

        @pl.when(nblk > 1)
        def _():
            fire_in(1, 1)

        def pair(q, _):
            for par in (0, 1):
                b = 2 * q + par

                @pl.when(b < nblk)
                def _():
                    @pl.when(b + 1 < nblk)
                    def _():
                        wait_in(b + 1, 1 - par)
                        idx_and_gather(1 - par)
                    for d in g_descs(par):
                        d.wait()
                    edges(b, par)

                    @pl.when(b + 2 < nblk)
                    def _():
                        fire_in(b + 2, par)
            return 0
        lax.fori_loop(0, (nblk + 1) >> 1, pair, 0)

        fb = nbase * wf
        pltpu.sync_copy(asum, ssum_h.at[pl.ds(fb, CH * wf)])
        pltpu.sync_copy(asq, ssq_h.at[pl.ds(fb, CH * wf)])
        pltpu.sync_copy(amn, smn_h.at[pl.ds(fb, CH * wf)])
        pltpu.sync_copy(amx, smx_h.at[pl.ds(fb, CH * wf)])
        return 0
    lax.fori_loop(0, NCHUNK, chunk, 0)


def _edge_stats(bp, pidx, eas, choff, w, wf):
    f = pl.kernel(
        functools.partial(_edge_body, wf),
        out_type=(
            jax.ShapeDtypeStruct((NP * wf,), jnp.float32),
            jax.ShapeDtypeStruct((NP * wf,), jnp.float32),
            jax.ShapeDtypeStruct((NP * wf,), jnp.float32),
            jax.ShapeDtypeStruct((NP * wf,), jnp.float32),
        ),
        mesh=_mesh(),
        compiler_params=_SC_PARAMS,
        scratch_types=[
            pltpu.VMEM((BG + 16,), jnp.int32),
            pltpu.VMEM((BG + 16,), jnp.int32),
            pltpu.VMEM((BG + 16,), jnp.float32),
            pltpu.VMEM((BG + 16,), jnp.float32),
            pltpu.VMEM((BG // 128, 128), jnp.int32),
            pltpu.VMEM((BG // 128, 128), jnp.int32),
            pltpu.VMEM((BG, wf), jnp.float32),
            pltpu.VMEM((BG, wf), jnp.float32),
            pltpu.VMEM((CH * wf,), jnp.float32),
            pltpu.VMEM((CH * wf,), jnp.float32),
            pltpu.VMEM((CH * wf,), jnp.float32),
            pltpu.VMEM((CH * wf,), jnp.float32),
            pltpu.VMEM((wf,), jnp.float32),
            pltpu.VMEM((32,), jnp.int32),
            pltpu.SemaphoreType.DMA,
            pltpu.SemaphoreType.DMA,
            pltpu.SemaphoreType.DMA,
            pltpu.SemaphoreType.DMA,
        ],
    )
    s, q, mn, mx = f(bp, pidx, eas, choff, w)
    rs = lambda t: t.reshape(NP, wf)
    return rs(s), rs(q), rs(mn), rs(mx)


NB = 256
NGRID = NP // NB


def _pre1_body(x_ref, wi_ref, wj_ref, c_ref, a_ref, b_ref):
    x = x_ref[...]
    a_ref[...] = jnp.dot(x, wi_ref[...],
                         preferred_element_type=jnp.float32) + c_ref[...]
    b_ref[...] = jnp.dot(x, wj_ref[...], preferred_element_type=jnp.float32)


def _pre1(xp, wi, wj, c):
    wf = wi.shape[1]
    return pl.pallas_call(
        _pre1_body,
        grid=(NGRID,),
        in_specs=[
            pl.BlockSpec((NB, xp.shape[1]), lambda i: (i, 0)),
            pl.BlockSpec((wi.shape[0], wf), lambda i: (0, 0)),
            pl.BlockSpec((wj.shape[0], wf), lambda i: (0, 0)),
            pl.BlockSpec((1, wf), lambda i: (0, 0)),
        ],
        out_specs=[
            pl.BlockSpec((NB, wf), lambda i: (i, 0)),
            pl.BlockSpec((NB, wf), lambda i: (i, 0)),
        ],
        out_shape=[
            jax.ShapeDtypeStruct((NP, wf), jnp.float32),
            jax.ShapeDtypeStruct((NP, wf), jnp.float32),
        ],
    )(xp, wi, wj, c)


def _post_math(h, u, ssum, ssq, smn, smx, d, wpp, bpost, wlin, blin, wf):
    degc = jnp.maximum(d, 1.0)
    nz = (d > 0.0).astype(jnp.float32)
    sm = ssum / degc
    mean = sm + u * nz
    var = ssq / degc - sm * sm
    std = jnp.sqrt(jnp.maximum(var, 0.0) + 1e-5)
    mn = (smn + u) * nz
    mx = (smx + u) * nz
    logd = jnp.log(degc + 1.0)
    s2 = logd * (1.0 / AVG_DEG_LOG)
    s3 = AVG_DEG_LOG / logd
    o = jnp.dot(h, wpp[0:wf], preferred_element_type=jnp.float32)
    for k, p in enumerate((mean, mn, mx, std)):
        o += jnp.dot(p, wpp[(1 + k) * wf:(2 + k) * wf],
                     preferred_element_type=jnp.float32)
        o += jnp.dot(p * s2, wpp[(5 + k) * wf:(6 + k) * wf],
                     preferred_element_type=jnp.float32)
        o += jnp.dot(p * s3, wpp[(9 + k) * wf:(10 + k) * wf],
                     preferred_element_type=jnp.float32)
    o = o + bpost
    return jnp.dot(o, wlin, preferred_element_type=jnp.float32) + blin


def _post_body(wf, h_ref, u_ref, ssum_ref, ssq_ref, smn_ref, smx_ref, d_ref,
               wpp_ref, bpost_ref, wlin_ref, blin_ref,
               win_ref, cn_ref, wjn_ref,
               h2_ref, an_ref, bn_ref):
    o = _post_math(h_ref[...], u_ref[...], ssum_ref[...], ssq_ref[...],
                   smn_ref[...], smx_ref[...], d_ref[...], wpp_ref[...],
                   bpost_ref[...], wlin_ref[...], blin_ref[...], wf)
    o = jnp.where(o > 0, o, 0.01 * o)
    h2_ref[...] = o
    an_ref[...] = jnp.dot(o, win_ref[...],
                          preferred_element_type=jnp.float32) + cn_ref[...]
    bn_ref[...] = jnp.dot(o, wjn_ref[...],
                          preferred_element_type=jnp.float32)


def _post_pre(h, u, ssum, ssq, smn, smx, deg2, wpp, bpost, wlin, blin,
              win, cn, wjn, wf):
    wfn = win.shape[1]
    return pl.pallas_call(
        functools.partial(_post_body, wf),
        grid=(NGRID,),
        in_specs=[
            pl.BlockSpec((NB, h.shape[1]), lambda i: (i, 0)),
            pl.BlockSpec((NB, wf), lambda i: (i, 0)),
            pl.BlockSpec((NB, wf), lambda i: (i, 0)),
            pl.BlockSpec((NB, wf), lambda i: (i, 0)),
            pl.BlockSpec((NB, wf), lambda i: (i, 0)),
            pl.BlockSpec((NB, wf), lambda i: (i, 0)),
            pl.BlockSpec((NB, 1), lambda i: (i, 0)),
            pl.BlockSpec(wpp.shape, lambda i: (0, 0)),
            pl.BlockSpec((1, HO), lambda i: (0, 0)),
            pl.BlockSpec((HO, HO), lambda i: (0, 0)),
            pl.BlockSpec((1, HO), lambda i: (0, 0)),
            pl.BlockSpec((HO, wfn), lambda i: (0, 0)),
            pl.BlockSpec((1, wfn), lambda i: (0, 0)),
            pl.BlockSpec((HO, wfn), lambda i: (0, 0)),
        ],
        out_specs=[
            pl.BlockSpec((NB, HO), lambda i: (i, 0)),
            pl.BlockSpec((NB, wfn), lambda i: (i, 0)),
            pl.BlockSpec((NB, wfn), lambda i: (i, 0)),
        ],
        out_shape=[
            jax.ShapeDtypeStruct((NP, HO), jnp.float32),
            jax.ShapeDtypeStruct((NP, wfn), jnp.float32),
            jax.ShapeDtypeStruct((NP, wfn), jnp.float32),
        ],
    )(h, u, ssum, ssq, smn, smx, deg2, wpp, bpost, wlin, blin, win, cn, wjn)


def _final_body(wf, h_ref, u_ref, ssum_ref, ssq_ref, smn_ref, smx_ref, d_ref,
                wpp_ref, bpost_ref, wlin_ref, blin_ref, bt_ref,
                w1_ref, b1_ref, w2_ref, b2_ref,
                out_ref, pacc, cacc):
    i = pl.program_id(0)

    @pl.when(i == 0)
    def _():
        pacc[...] = jnp.zeros_like(pacc)
        cacc[...] = jnp.zeros_like(cacc)

    o = _post_math(h_ref[...], u_ref[...], ssum_ref[...], ssq_ref[...],
                   smn_ref[...], smx_ref[...], d_ref[...], wpp_ref[...],
                   bpost_ref[...], wlin_ref[...], blin_ref[...], wf)
    bt = bt_ref[...]
    oh = (bt == lax.broadcasted_iota(jnp.int32, (NB, NG), 1)).astype(
        jnp.float32)
    pacc[...] += lax.dot_general(oh, o, (((0,), (0,)), ((), ())),
                                 preferred_element_type=jnp.float32)
    cacc[...] += lax.dot_general(
        oh, jnp.ones((NB, 8), jnp.float32), (((0,), (0,)), ((), ())),
        preferred_element_type=jnp.float32)

    @pl.when(i == NGRID - 1)
    def _():
        cnt = jnp.maximum(cacc[...][:, 0:1], 1.0)
        pooled = pacc[...] / cnt
        z = jnp.dot(pooled, w1_ref[...],
                    preferred_element_type=jnp.float32) + b1_ref[...]
        z = jnp.maximum(z, 0.0)
        out_ref[...] = jnp.dot(z, w2_ref[...],
                               preferred_element_type=jnp.float32) + b2_ref[...]


def _final(h, u, ssum, ssq, smn, smx, deg2, wpp, bpost, wlin, blin, batch2,
           w1, b1, w2p, b2p, wf):
    return pl.pallas_call(
        functools.partial(_final_body, wf),
        grid=(NGRID,),
        in_specs=[
            pl.BlockSpec((NB, h.shape[1]), lambda i: (i, 0)),
            pl.BlockSpec((NB, wf), lambda i: (i, 0)),
            pl.BlockSpec((NB, wf), lambda i: (i, 0)),
            pl.BlockSpec((NB, wf), lambda i: (i, 0)),
            pl.BlockSpec((NB, wf), lambda i: (i, 0)),
            pl.BlockSpec((NB, wf), lambda i: (i, 0)),
            pl.BlockSpec((NB, 1), lambda i: (i, 0)),
            pl.BlockSpec(wpp.shape, lambda i: (0, 0)),
            pl.BlockSpec((1, HO), lambda i: (0, 0)),
            pl.BlockSpec((HO, HO), lambda i: (0, 0)),
            pl.BlockSpec((1, HO), lambda i: (0, 0)),
            pl.BlockSpec((NB, 1), lambda i: (i, 0)),
            pl.BlockSpec((HO, 32), lambda i: (0, 0)),
            pl.BlockSpec((1, 32), lambda i: (0, 0)),
            pl.BlockSpec((32, 128), lambda i: (0, 0)),
            pl.BlockSpec((1, 128), lambda i: (0, 0)),
        ],
        out_specs=[pl.BlockSpec((NG, 128), lambda i: (0, 0))],
        out_shape=[jax.ShapeDtypeStruct((NG, 128), jnp.float32)],
        scratch_shapes=[
            pltpu.VMEM((NG, HO), jnp.float32),
            pltpu.VMEM((NG, 8), jnp.float32),
        ],
    )(h, u, ssum, ssq, smn, smx, deg2, wpp, bpost, wlin, blin, batch2,
      w1, b1, w2p, b2p)[0]


def _prep_conv(p, f_real, wf):
    wpre = p["Wpre"]
    wi = wpre[:f_real]
    wj = wpre[f_real:2 * f_real]
    we2 = wpre[2 * f_real:3 * f_real]
    w = p["We"][0] @ we2
    c = p["be"] @ we2 + p["bpre"]
    pad = wf - f_real
    wi = jnp.pad(wi, ((0, pad), (0, pad)))
    wj = jnp.pad(wj, ((0, pad), (0, pad)))
    w = jnp.pad(w, (0, pad))
    c = jnp.pad(c, (0, pad))
    wpost = p["Wpost"]
    blocks = [jnp.pad(wpost[k * f_real:(k + 1) * f_real], ((0, pad), (0, 0)))
              for k in range(13)]
    wpp = jnp.concatenate(blocks, axis=0)
    return (wi, wj, c[None, :], w, wpp, p["bpost"][None, :],
            p["Wlin"], p["blin"][None, :])


def kernel(x, edge_index, edge_attr, batch, conv1, conv2, conv3, lin):
    src = edge_index[0].astype(jnp.int32)
    dst = edge_index[1].astype(jnp.int32)
    ea = edge_attr[:, 0]

    wi1, wj1, c1, w1v, wpp1, bp1, wl1, bl1 = _prep_conv(conv1, 7, 16)
    wi2, wj2, c2, w2v, wpp2, bp2, wl2, bl2 = _prep_conv(conv2, 64, 64)
    wi3, wj3, c3, w3v, wpp3, bp3, wl3, bl3 = _prep_conv(conv3, 64, 64)

    xp = jnp.pad(x, ((0, NP - N), (0, 16 - 7)))
    batch2 = jnp.pad(batch.astype(jnp.int32), (0, NP - N),
                     constant_values=NG)[:, None]
    w2p = jnp.pad(lin["W2"], ((0, 0), (0, 128 - NCLS)))
    b2p = jnp.pad(lin["b2"], (0, 128 - NCLS))[None, :]

    pidx, eas, choff, deg = _bucket(dst, src, ea)
    deg2 = deg[:, None]

    a1, b1 = _pre1(xp, wi1, wj1, c1)
    s1, q1, mn1, mx1 = _edge_stats(b1, pidx, eas, choff, w1v, 16)
    h2, a2, b2 = _post_pre(xp, a1, s1, q1, mn1, mx1, deg2, wpp1, bp1, wl1,
                           bl1, wi2, c2, wj2, 16)
    s2, q2, mn2, mx2 = _edge_stats(b2, pidx, eas, choff, w2v, 64)
    h3, a3, b3 = _post_pre(h2, a2, s2, q2, mn2, mx2, deg2, wpp2, bp2, wl2,
                           bl2, wi3, c3, wj3, 64)
    s3, q3, mn3, mx3 = _edge_stats(b3, pidx, eas, choff, w3v, 64)
    out = _final(h3, a3, s3, q3, mn3, mx3, deg2, wpp3, bp3, wl3, bl3, batch2,
                 lin["W1"], lin["b1"][None, :], w2p, b2p, 64)
    return out[:, :NCLS]


# speedup vs baseline: 6.8379x; 1.4619x over previous
import functools
import math

import jax
import jax.numpy as jnp
from jax import lax
from jax.experimental import pallas as pl
from jax.experimental.pallas import tpu as pltpu
from jax.experimental.pallas import tpu_sc as plsc

N = 50000
E = 800000
NG = 128
NCLS = 5
HO = 64

NC = 2
NS = 16
NW = NC * NS
RANGE = 1664
NP = NW * RANGE
CH = 128
NCHUNK = RANGE // CH
CAP = 28416
HCAP = CAP // 2
BA = 1600
NBLKA = E // BA
BG = 512

AVG_DEG_LOG = math.log(17.0)
FMAX = 3.0e38

_SC_PARAMS = pltpu.CompilerParams(needs_layout_passes=False,
                                  use_tc_tiling_on_sc=False)


@functools.cache
def _mesh():
    return plsc.VectorSubcoreMesh(core_axis_name="c", subcore_axis_name="s")


def _wid():
    return lax.axis_index("s") * NC + lax.axis_index("c")


def _splat(v):
    return jnp.full((16,), v, jnp.int32)


def _sget(ref, i):
    return plsc.load_gather(ref, indices=[_splat(i)])[0]


def _sset(ref, i, v):
    plsc.store_scatter(ref, [_splat(i)], jnp.full((16,), v))


def _bucket_body(dst_h, src_h, ea_h, pidx_h, eas_h, choff_h, deg_h,
                 db0, db1, sb0, sb1, eb0, eb1, stg_p, stg_e,
                 hist16, off16, deg16, pos2, out_p, out_e,
                 degf, chof, sem, semf0, semf1):
    wid = _wid()
    lo = wid * RANGE
    lanes = lax.iota(jnp.int32, 16)
    ones16 = jnp.ones((16,), jnp.int32)

    def z16(v, _):
        stg_p[pl.ds(v * 16, 16)] = jnp.zeros((16,), jnp.int32)
        return 0
    lax.fori_loop(0, CAP // 16, z16, 0)

    def zh(v, _):
        hist16[pl.ds(v * 16, 16)] = jnp.zeros((16,), jnp.int32)
        return 0
    lax.fori_loop(0, 224 // 16, zh, 0)

    def zp(v, _):
        pos2[v >> 3, pl.ds((v & 7) * 16, 16)] = (
            lax.iota(jnp.int32, 16) + v * 16 + 128)
        return 0
    lax.fori_loop(0, CAP // 16, zp, 0)

    dbufs, sbufs, ebufs = (db0, db1), (sb0, sb1), (eb0, eb1)
    semfs = (semf0, semf1)

    def a_descs(b, par):
        bb = b + wid * (NBLKA // NW)
        bb = jnp.where(bb >= NBLKA, bb - NBLKA, bb)
        base = pl.multiple_of(bb * BA, 8)
        return (pltpu.make_async_copy(dst_h.at[pl.ds(base, BA)],
                                      dbufs[par], semfs[par]),
                pltpu.make_async_copy(src_h.at[pl.ds(base, BA)],
                                      sbufs[par], semfs[par]),
                pltpu.make_async_copy(ea_h.at[pl.ds(base, BA)],
                                      ebufs[par], semfs[par]))

    def a_fire(b, par):
        for dsc in a_descs(b, par):
            dsc.start()

    def a_wait(b, par):
        for dsc in a_descs(b, par):
            dsc.wait()

    def a_compute(par, cur):
        dbuf, sbuf, ebuf = dbufs[par], sbufs[par], ebufs[par]

        def vec(i, cur):
            for k in range(4):
                o = i * 64 + k * 16
                d = dbuf[pl.ds(o, 16)]
                s = sbuf[pl.ds(o, 16)]
                a = ebuf[pl.ds(o, 16)]
                msk = (d >= lo) & (d < lo + RANGE)
                pk = ((d - lo) << 16) | s
                plsc.store_compressed(stg_p.at[pl.ds(cur, 16)], pk, mask=msk)
                plsc.store_compressed(stg_e.at[pl.ds(cur, 16)], a, mask=msk)
                cur = cur + plsc.all_reduce_population_count(msk)[0]
            return cur
        return lax.fori_loop(0, BA // 64, vec, cur)

    def pairblk(q, cur):
        b0 = 2 * q
        a_fire(b0 + 1, 1)
        a_wait(b0, 0)
        cur = a_compute(0, cur)

        @pl.when(b0 + 2 < NBLKA)
        def _():
            a_fire(b0 + 2, 0)
        a_wait(b0 + 1, 1)
        cur = a_compute(1, cur)
        return cur
    with jax.named_scope("bkt_filter"):
        a_fire(0, 0)
        cnt = lax.fori_loop(0, NBLKA // 2, pairblk, jnp.int32(0))

    nfull = cnt >> 4
    tail = cnt & 15
    tm = lanes < tail

    def hv(v, _):
        pk = stg_p[pl.ds(v * 16, 16)]
        plsc.addupdate_scatter(hist16, [(pk >> 23) * 16 + lanes], ones16)
        return 0
    with jax.named_scope("bkt_hist"):
        lax.fori_loop(0, nfull, hv, 0)
    pkt = stg_p[pl.ds(nfull * 16, 16)]
    plsc.addupdate_scatter(hist16, [(pkt >> 23) * 16 + lanes], ones16,
                           mask=tm)

    def pre(cc, run):
        start = (run + 7) & ~7
        _sset(chof, cc, start)
        sl = pl.ds(cc * 16, 16)
        hv16 = hist16[sl]
        off16[sl] = plsc.cumsum(hv16) - hv16 + start
        tot = jnp.sum(hv16)
        _sset(chof, 16 + cc, tot)
        return start + tot
    lax.fori_loop(0, NCHUNK, pre, jnp.int32(0))
    _sset(chof, 15, cnt)
    pltpu.sync_copy(chof, choff_h.at[wid])

    def qv(v, _):
        pk = stg_p[pl.ds(v * 16, 16)]
        idxv = (pk >> 23) * 16 + lanes
        p = plsc.load_gather(off16, [idxv])
        plsc.store_scatter(off16, [idxv], p + 1)
        pos2[v >> 3, pl.ds((v & 7) * 16, 16)] = p
        return 0
    with jax.named_scope("bkt_pos"):
        lax.fori_loop(0, nfull, qv, 0)
    idxt = (pkt >> 23) * 16 + lanes
    pt = plsc.load_gather(off16, [idxt], mask=tm)
    plsc.store_scatter(off16, [idxt], pt + 1, mask=tm)
    tsl = pl.ds((nfull * 16) & 127, 16)
    pos2[nfull >> 3, tsl] = jnp.where(tm, pt, pos2[nfull >> 3, tsl])

    with jax.named_scope("bkt_scatter"):
        for h in (0, 1):
            h0 = h * HCAP

            def pv(v, _):
                pk = stg_p[pl.ds(v * 16, 16)]
                ae = stg_e[pl.ds(v * 16, 16)]
                p = pos2[v >> 3, pl.ds((v & 7) * 16, 16)]
                msk = (p >= h0) & (p < h0 + HCAP)
                pi = jnp.clip(p - h0, 0, HCAP - 1)
                plsc.store_scatter(out_p, [pi], pk, mask=msk)
                plsc.store_scatter(out_e, [pi], ae, mask=msk)
                return 0
            lax.fori_loop(0, (cnt + 15) >> 4, pv, 0)
            pltpu.sync_copy(out_p, pidx_h.at[wid].at[pl.ds(h0, HCAP)])
            pltpu.sync_copy(out_e, eas_h.at[wid].at[pl.ds(h0, HCAP)])

    def dchunk(c, _):
        def zd(v, _):
            deg16[pl.ds(v * 16, 16)] = jnp.zeros((16,), jnp.int32)
            return 0
        lax.fori_loop(0, CH, zd, 0)

        def sv(v, _):
            pk = stg_p[pl.ds(v * 16, 16)]
            msk = ((pk >> 23) == c) & ((v * 16 + lanes) < cnt)
            idxv = ((pk >> 16) - c * CH) * 16 + lanes
            idxv = jnp.clip(idxv, 0, CH * 16 - 1)
            plsc.addupdate_scatter(deg16, [idxv], ones16, mask=msk)
            return 0
        lax.fori_loop(0, (cnt + 15) >> 4, sv, 0)

        def rd(n, _):
            s = jnp.sum(deg16[pl.ds(n * 16, 16)])
            _sset(degf, c * CH + n, s.astype(jnp.float32))
            return 0
        lax.fori_loop(0, CH, rd, 0)
        return 0
    with jax.named_scope("bkt_deg"):
        lax.fori_loop(0, NCHUNK, dchunk, 0)
    pltpu.sync_copy(degf, deg_h.at[pl.ds(lo, RANGE)])


def _bucket(dst, src, ea):
    f = pl.kernel(
        _bucket_body,
        out_type=(
            jax.ShapeDtypeStruct((NW, CAP), jnp.int32),
            jax.ShapeDtypeStruct((NW, CAP), jnp.float32),
            jax.ShapeDtypeStruct((NW, 32), jnp.int32),
            jax.ShapeDtypeStruct((NP,), jnp.float32),
        ),
        mesh=_mesh(),
        compiler_params=_SC_PARAMS,
        scratch_types=[
            pltpu.VMEM((BA,), jnp.int32),
            pltpu.VMEM((BA,), jnp.int32),
            pltpu.VMEM((BA,), jnp.int32),
            pltpu.VMEM((BA,), jnp.int32),
            pltpu.VMEM((BA,), jnp.float32),
            pltpu.VMEM((BA,), jnp.float32),
            pltpu.VMEM((CAP + 16,), jnp.int32),
            pltpu.VMEM((CAP + 16,), jnp.float32),
            pltpu.VMEM((224,), jnp.int32),
            pltpu.VMEM((224,), jnp.int32),
            pltpu.VMEM((CH * 16,), jnp.int32),
            pltpu.VMEM((CAP // 128, 128), jnp.int32),
            pltpu.VMEM((HCAP,), jnp.int32),
            pltpu.VMEM((HCAP,), jnp.float32),
            pltpu.VMEM((RANGE,), jnp.float32),
            pltpu.VMEM((32,), jnp.int32),
            pltpu.SemaphoreType.DMA,
            pltpu.SemaphoreType.DMA,
            pltpu.SemaphoreType.DMA,
        ],
    )
    return f(dst, src, ea)


def _edge_body(wf, bp_h, pidx_h, eas_h, choff_h, w_h,
               ssum_h, ssq_h, smn_h, smx_h,
               pb0, pb1, eb0, eb1, ix0, ix1, rw0, rw1,
               asum, asq, amn, amx, wv, chof,
               smi0, smi1, smg0, smg1):
    wid = _wid()
    nb = wid * RANGE
    pltpu.sync_copy(choff_h.at[wid], chof)
    pltpu.sync_copy(w_h, wv)
    nj = wf // 16
    pbufs = (pb0, pb1)
    ebufs = (eb0, eb1)
    ixs = (ix0, ix1)
    rws = (rw0, rw1)
    smis = (smi0, smi1)
    smgs = (smg0, smg1)

    def chunk(c, _):
        e0 = _sget(chof, c)
        ec = _sget(chof, 16 + c)
        nbase = nb + c * CH

        def init_v(v, _):
            sl = pl.ds(v * 16, 16)
            asum[sl] = jnp.zeros((16,), jnp.float32)
            asq[sl] = jnp.zeros((16,), jnp.float32)
            amn[sl] = jnp.full((16,), FMAX, jnp.float32)
            amx[sl] = jnp.full((16,), -FMAX, jnp.float32)
            return 0
        lax.fori_loop(0, CH * wf // 16, init_v, 0)

        nblk = (ec + BG - 1) >> 9

        def in_descs(b, par):
            base = pl.multiple_of(e0 + b * BG, 8)
            d1 = pltpu.make_async_copy(pidx_h.at[wid].at[pl.ds(base, BG)],
                                       pbufs[par].at[pl.ds(0, BG)], smis[par])
            d2 = pltpu.make_async_copy(eas_h.at[wid].at[pl.ds(base, BG)],
                                       ebufs[par].at[pl.ds(0, BG)], smis[par])
            return d1, d2

        def g_descs(par):
            return [pltpu.make_async_copy(bp_h.at[ixs[par].at[j]],
                                          rws[par].at[pl.ds(j * 128, 128)],
                                          smgs[par])
                    for j in range(BG // 128)]

        def fire_in(b, par):
            d1, d2 = in_descs(b, par)
            d1.start()
            d2.start()

        def wait_in(b, par):
            d1, d2 = in_descs(b, par)
            d1.wait()
            d2.wait()

        def idx_and_gather(par):
            def vi(v, _):
                s = pbufs[par][pl.ds(v * 16, 16)] & 0xFFFF
                ixs[par][v >> 3, pl.ds((v & 7) * 16, 16)] = jnp.minimum(
                    s, jnp.int32(N - 1))
                return 0
            lax.fori_loop(0, BG // 16, vi, 0)
            for d in g_descs(par):
                d.start()

        def edges(b, par):
            nrem = jnp.minimum(jnp.int32(BG), ec - b * BG)
            pbuf, ebuf, rows = pbufs[par], ebufs[par], rws[par]
            cbase = c * (CH * wf)

            def one(e):
                pk = pbuf[pl.ds(e, 16)][0]
                off = (pk >> 16) * wf - cbase
                a = ebuf[pl.ds(e, 16)][0]
                for j in range(nj):
                    sl = pl.ds(off + j * 16, 16)
                    t = rows[e, pl.ds(j * 16, 16)] + a * wv[pl.ds(j * 16, 16)]
                    plsc.addupdate(asum.at[sl], t)
                    plsc.addupdate(asq.at[sl], t * t)
                    amn[sl] = jnp.minimum(amn[sl], t)
                    amx[sl] = jnp.maximum(amx[sl], t)

            def epair(q, _):
                one(2 * q)

                @pl.when(2 * q + 1 < nrem)
                def _():
                    one(2 * q + 1)
                return 0
            lax.fori_loop(0, (nrem + 1) >> 1, epair, 0)

        @pl.when(nblk > 0)
        def _():
            fire_in(0, 0)
            wait_in(0, 0)
            idx_and_gather(0)

        @pl.when(nblk > 1)
        def _():
            fire_in(1, 1)

        def pair(q, _):
            for par in (0, 1):
                b = 2 * q + par

                @pl.when(b < nblk)
                def _():
                    @pl.when(b + 1 < nblk)
                    def _():
                        wait_in(b + 1, 1 - par)
                        idx_and_gather(1 - par)
                    for d in g_descs(par):
                        d.wait()
                    edges(b, par)

                    @pl.when(b + 2 < nblk)
                    def _():
                        fire_in(b + 2, par)
            return 0
        lax.fori_loop(0, (nblk + 1) >> 1, pair, 0)

        fb = nbase * wf
        pltpu.sync_copy(asum, ssum_h.at[pl.ds(fb, CH * wf)])
        pltpu.sync_copy(asq, ssq_h.at[pl.ds(fb, CH * wf)])
        pltpu.sync_copy(amn, smn_h.at[pl.ds(fb, CH * wf)])
        pltpu.sync_copy(amx, smx_h.at[pl.ds(fb, CH * wf)])
        return 0
    lax.fori_loop(0, NCHUNK, chunk, 0)


def _edge_stats(bp, pidx, eas, choff, w, wf):
    f = pl.kernel(
        functools.partial(_edge_body, wf),
        out_type=(
            jax.ShapeDtypeStruct((NP * wf,), jnp.float32),
            jax.ShapeDtypeStruct((NP * wf,), jnp.float32),
            jax.ShapeDtypeStruct((NP * wf,), jnp.float32),
            jax.ShapeDtypeStruct((NP * wf,), jnp.float32),
        ),
        mesh=_mesh(),
        compiler_params=_SC_PARAMS,
        scratch_types=[
            pltpu.VMEM((BG + 16,), jnp.int32),
            pltpu.VMEM((BG + 16,), jnp.int32),
            pltpu.VMEM((BG + 16,), jnp.float32),
            pltpu.VMEM((BG + 16,), jnp.float32),
            pltpu.VMEM((BG // 128, 128), jnp.int32),
            pltpu.VMEM((BG // 128, 128), jnp.int32),
            pltpu.VMEM((BG, wf), jnp.float32),
            pltpu.VMEM((BG, wf), jnp.float32),
            pltpu.VMEM((CH * wf,), jnp.float32),
            pltpu.VMEM((CH * wf,), jnp.float32),
            pltpu.VMEM((CH * wf,), jnp.float32),
            pltpu.VMEM((CH * wf,), jnp.float32),
            pltpu.VMEM((wf,), jnp.float32),
            pltpu.VMEM((32,), jnp.int32),
            pltpu.SemaphoreType.DMA,
            pltpu.SemaphoreType.DMA,
            pltpu.SemaphoreType.DMA,
            pltpu.SemaphoreType.DMA,
        ],
    )
    s, q, mn, mx = f(bp, pidx, eas, choff, w)
    rs = lambda t: t.reshape(NP, wf)
    return rs(s), rs(q), rs(mn), rs(mx)


NB = 512
NGRID = NP // NB


def _pre1_body(x_ref, wi_ref, wj_ref, c_ref, a_ref, b_ref):
    x = x_ref[...]
    a_ref[...] = jnp.dot(x, wi_ref[...],
                         preferred_element_type=jnp.float32) + c_ref[...]
    b_ref[...] = jnp.dot(x, wj_ref[...], preferred_element_type=jnp.float32)


def _pre1(xp, wi, wj, c):
    wf = wi.shape[1]
    return pl.pallas_call(
        _pre1_body,
        grid=(NGRID,),
        in_specs=[
            pl.BlockSpec((NB, xp.shape[1]), lambda i: (i, 0)),
            pl.BlockSpec((wi.shape[0], wf), lambda i: (0, 0)),
            pl.BlockSpec((wj.shape[0], wf), lambda i: (0, 0)),
            pl.BlockSpec((1, wf), lambda i: (0, 0)),
        ],
        out_specs=[
            pl.BlockSpec((NB, wf), lambda i: (i, 0)),
            pl.BlockSpec((NB, wf), lambda i: (i, 0)),
        ],
        out_shape=[
            jax.ShapeDtypeStruct((NP, wf), jnp.float32),
            jax.ShapeDtypeStruct((NP, wf), jnp.float32),
        ],
    )(xp, wi, wj, c)


def _post_math(h, u, ssum, ssq, smn, smx, d, wpp, bpost, wlin, blin, wf):
    degc = jnp.maximum(d, 1.0)
    nz = (d > 0.0).astype(jnp.float32)
    sm = ssum / degc
    mean = sm + u * nz
    var = ssq / degc - sm * sm
    std = jnp.sqrt(jnp.maximum(var, 0.0) + 1e-5)
    mn = (smn + u) * nz
    mx = (smx + u) * nz
    logd = jnp.log(degc + 1.0)
    s2 = logd * (1.0 / AVG_DEG_LOG)
    s3 = AVG_DEG_LOG / logd
    o = jnp.dot(h, wpp[0:wf], preferred_element_type=jnp.float32)
    for k, p in enumerate((mean, mn, mx, std)):
        o += jnp.dot(p, wpp[(1 + k) * wf:(2 + k) * wf],
                     preferred_element_type=jnp.float32)
        o += jnp.dot(p * s2, wpp[(5 + k) * wf:(6 + k) * wf],
                     preferred_element_type=jnp.float32)
        o += jnp.dot(p * s3, wpp[(9 + k) * wf:(10 + k) * wf],
                     preferred_element_type=jnp.float32)
    o = o + bpost
    return jnp.dot(o, wlin, preferred_element_type=jnp.float32) + blin


def _post_body(wf, h_ref, u_ref, ssum_ref, ssq_ref, smn_ref, smx_ref, d_ref,
               wpp_ref, bpost_ref, wlin_ref, blin_ref,
               win_ref, cn_ref, wjn_ref,
               h2_ref, an_ref, bn_ref):
    o = _post_math(h_ref[...], u_ref[...], ssum_ref[...], ssq_ref[...],
                   smn_ref[...], smx_ref[...], d_ref[...], wpp_ref[...],
                   bpost_ref[...], wlin_ref[...], blin_ref[...], wf)
    o = jnp.where(o > 0, o, 0.01 * o)
    h2_ref[...] = o
    an_ref[...] = jnp.dot(o, win_ref[...],
                          preferred_element_type=jnp.float32) + cn_ref[...]
    bn_ref[...] = jnp.dot(o, wjn_ref[...],
                          preferred_element_type=jnp.float32)


def _post_pre(h, u, ssum, ssq, smn, smx, deg2, wpp, bpost, wlin, blin,
              win, cn, wjn, wf):
    wfn = win.shape[1]
    return pl.pallas_call(
        functools.partial(_post_body, wf),
        grid=(NGRID,),
        in_specs=[
            pl.BlockSpec((NB, h.shape[1]), lambda i: (i, 0)),
            pl.BlockSpec((NB, wf), lambda i: (i, 0)),
            pl.BlockSpec((NB, wf), lambda i: (i, 0)),
            pl.BlockSpec((NB, wf), lambda i: (i, 0)),
            pl.BlockSpec((NB, wf), lambda i: (i, 0)),
            pl.BlockSpec((NB, wf), lambda i: (i, 0)),
            pl.BlockSpec((NB, 1), lambda i: (i, 0)),
            pl.BlockSpec(wpp.shape, lambda i: (0, 0)),
            pl.BlockSpec((1, HO), lambda i: (0, 0)),
            pl.BlockSpec((HO, HO), lambda i: (0, 0)),
            pl.BlockSpec((1, HO), lambda i: (0, 0)),
            pl.BlockSpec((HO, wfn), lambda i: (0, 0)),
            pl.BlockSpec((1, wfn), lambda i: (0, 0)),
            pl.BlockSpec((HO, wfn), lambda i: (0, 0)),
        ],
        out_specs=[
            pl.BlockSpec((NB, HO), lambda i: (i, 0)),
            pl.BlockSpec((NB, wfn), lambda i: (i, 0)),
            pl.BlockSpec((NB, wfn), lambda i: (i, 0)),
        ],
        out_shape=[
            jax.ShapeDtypeStruct((NP, HO), jnp.float32),
            jax.ShapeDtypeStruct((NP, wfn), jnp.float32),
            jax.ShapeDtypeStruct((NP, wfn), jnp.float32),
        ],
    )(h, u, ssum, ssq, smn, smx, deg2, wpp, bpost, wlin, blin, win, cn, wjn)


def _final_body(wf, h_ref, u_ref, ssum_ref, ssq_ref, smn_ref, smx_ref, d_ref,
                wpp_ref, bpost_ref, wlin_ref, blin_ref, bt_ref,
                w1_ref, b1_ref, w2_ref, b2_ref,
                out_ref, pacc, cacc):
    i = pl.program_id(0)

    @pl.when(i == 0)
    def _():
        pacc[...] = jnp.zeros_like(pacc)
        cacc[...] = jnp.zeros_like(cacc)

    o = _post_math(h_ref[...], u_ref[...], ssum_ref[...], ssq_ref[...],
                   smn_ref[...], smx_ref[...], d_ref[...], wpp_ref[...],
                   bpost_ref[...], wlin_ref[...], blin_ref[...], wf)
    bt = bt_ref[...]
    oh = (bt == lax.broadcasted_iota(jnp.int32, (NB, NG), 1)).astype(
        jnp.float32)
    pacc[...] += lax.dot_general(oh, o, (((0,), (0,)), ((), ())),
                                 preferred_element_type=jnp.float32)
    cacc[...] += lax.dot_general(
        oh, jnp.ones((NB, 8), jnp.float32), (((0,), (0,)), ((), ())),
        preferred_element_type=jnp.float32)

    @pl.when(i == NGRID - 1)
    def _():
        cnt = jnp.maximum(cacc[...][:, 0:1], 1.0)
        pooled = pacc[...] / cnt
        z = jnp.dot(pooled, w1_ref[...],
                    preferred_element_type=jnp.float32) + b1_ref[...]
        z = jnp.maximum(z, 0.0)
        out_ref[...] = jnp.dot(z, w2_ref[...],
                               preferred_element_type=jnp.float32) + b2_ref[...]


def _final(h, u, ssum, ssq, smn, smx, deg2, wpp, bpost, wlin, blin, batch2,
           w1, b1, w2p, b2p, wf):
    return pl.pallas_call(
        functools.partial(_final_body, wf),
        grid=(NGRID,),
        in_specs=[
            pl.BlockSpec((NB, h.shape[1]), lambda i: (i, 0)),
            pl.BlockSpec((NB, wf), lambda i: (i, 0)),
            pl.BlockSpec((NB, wf), lambda i: (i, 0)),
            pl.BlockSpec((NB, wf), lambda i: (i, 0)),
            pl.BlockSpec((NB, wf), lambda i: (i, 0)),
            pl.BlockSpec((NB, wf), lambda i: (i, 0)),
            pl.BlockSpec((NB, 1), lambda i: (i, 0)),
            pl.BlockSpec(wpp.shape, lambda i: (0, 0)),
            pl.BlockSpec((1, HO), lambda i: (0, 0)),
            pl.BlockSpec((HO, HO), lambda i: (0, 0)),
            pl.BlockSpec((1, HO), lambda i: (0, 0)),
            pl.BlockSpec((NB, 1), lambda i: (i, 0)),
            pl.BlockSpec((HO, 32), lambda i: (0, 0)),
            pl.BlockSpec((1, 32), lambda i: (0, 0)),
            pl.BlockSpec((32, 128), lambda i: (0, 0)),
            pl.BlockSpec((1, 128), lambda i: (0, 0)),
        ],
        out_specs=[pl.BlockSpec((NG, 128), lambda i: (0, 0))],
        out_shape=[jax.ShapeDtypeStruct((NG, 128), jnp.float32)],
        scratch_shapes=[
            pltpu.VMEM((NG, HO), jnp.float32),
            pltpu.VMEM((NG, 8), jnp.float32),
        ],
    )(h, u, ssum, ssq, smn, smx, deg2, wpp, bpost, wlin, blin, batch2,
      w1, b1, w2p, b2p)[0]


def _prep_conv(p, f_real, wf):
    wpre = p["Wpre"]
    wi = wpre[:f_real]
    wj = wpre[f_real:2 * f_real]
    we2 = wpre[2 * f_real:3 * f_real]
    w = p["We"][0] @ we2
    c = p["be"] @ we2 + p["bpre"]
    pad = wf - f_real
    wi = jnp.pad(wi, ((0, pad), (0, pad)))
    wj = jnp.pad(wj, ((0, pad), (0, pad)))
    w = jnp.pad(w, (0, pad))
    c = jnp.pad(c, (0, pad))
    wpost = p["Wpost"]
    blocks = [jnp.pad(wpost[k * f_real:(k + 1) * f_real], ((0, pad), (0, 0)))
              for k in range(13)]
    wpp = jnp.concatenate(blocks, axis=0)
    return (wi, wj, c[None, :], w, wpp, p["bpost"][None, :],
            p["Wlin"], p["blin"][None, :])


def kernel(x, edge_index, edge_attr, batch, conv1, conv2, conv3, lin):
    src = edge_index[0].astype(jnp.int32)
    dst = edge_index[1].astype(jnp.int32)
    ea = edge_attr[:, 0]

    wi1, wj1, c1, w1v, wpp1, bp1, wl1, bl1 = _prep_conv(conv1, 7, 16)
    wi2, wj2, c2, w2v, wpp2, bp2, wl2, bl2 = _prep_conv(conv2, 64, 64)
    wi3, wj3, c3, w3v, wpp3, bp3, wl3, bl3 = _prep_conv(conv3, 64, 64)

    xp = jnp.pad(x, ((0, NP - N), (0, 16 - 7)))
    batch2 = jnp.pad(batch.astype(jnp.int32), (0, NP - N),
                     constant_values=NG)[:, None]
    w2p = jnp.pad(lin["W2"], ((0, 0), (0, 128 - NCLS)))
    b2p = jnp.pad(lin["b2"], (0, 128 - NCLS))[None, :]

    pidx, eas, choff, deg = _bucket(dst, src, ea)
    deg2 = deg[:, None]

    a1, b1 = _pre1(xp, wi1, wj1, c1)
    s1, q1, mn1, mx1 = _edge_stats(b1, pidx, eas, choff, w1v, 16)
    h2, a2, b2 = _post_pre(xp, a1, s1, q1, mn1, mx1, deg2, wpp1, bp1, wl1,
                           bl1, wi2, c2, wj2, 16)
    s2, q2, mn2, mx2 = _edge_stats(b2, pidx, eas, choff, w2v, 64)
    h3, a3, b3 = _post_pre(h2, a2, s2, q2, mn2, mx2, deg2, wpp2, bp2, wl2,
                           bl2, wi3, c3, wj3, 64)
    s3, q3, mn3, mx3 = _edge_stats(b3, pidx, eas, choff, w3v, 64)
    out = _final(h3, a3, s3, q3, mn3, mx3, deg2, wpp3, bp3, wl3, bl3, batch2,
                 lin["W1"], lin["b1"][None, :], w2p, b2p, 64)
    return out[:, :NCLS]
